# Initial kernel scaffold; baseline (speedup 1.0000x reference)
import jax
import jax.numpy as jnp
from jax.experimental import pallas as pl


def kernel(x, edge_index, edge_attr, W1_0, W1_1, b1, W2_0, W2_1, b2):
    raise NotImplementedError("write your pallas kernel here")


# speedup vs baseline: 12.8055x; 12.8055x over previous
import functools

import jax
import jax.numpy as jnp
from jax import lax
from jax.experimental import pallas as pl
from jax.experimental.pallas import tpu as pltpu
from jax.experimental.pallas import tpu_sc as plsc

NC = 2
NS = 16
NW = NC * NS
LANES = 16
CHUNK = 128

F32 = jnp.float32
I32 = jnp.int32

_SC_PARAMS = pltpu.CompilerParams(use_tc_tiling_on_sc=False)


def _round_up(v, m):
    return (v + m - 1) // m * m


def _mesh():
    return plsc.VectorSubcoreMesh(core_axis_name="c", subcore_axis_name="s",
                                  num_cores=NC, num_subcores=NS)


def _splat(vec, e):
    idx = jnp.full((LANES,), e, I32)
    return lax.gather(
        vec, idx[:, None],
        lax.GatherDimensionNumbers(offset_dims=(), collapsed_slice_dims=(0,),
                                   start_index_map=(0,)),
        (1,), mode=lax.GatherScatterMode.PROMISE_IN_BOUNDS)


def _sc_deg(row_p, col_p, w_p, n_pad):
    e_pad = row_p.shape[0]
    per_tile = e_pad // NW
    n_chunks = per_tile // CHUNK
    stripe = n_pad // NS

    @functools.partial(
        pl.kernel,
        out_type=(jax.ShapeDtypeStruct((NC * n_pad, LANES), F32),
                  jax.ShapeDtypeStruct((e_pad,), F32)),
        mesh=_mesh(),
        scratch_types=[
            pltpu.VMEM_SHARED((n_pad, LANES), F32),
            pltpu.VMEM((CHUNK,), I32),
            pltpu.VMEM((CHUNK,), I32),
            pltpu.VMEM((CHUNK,), F32),
            pltpu.VMEM((CHUNK,), F32),
            pltpu.VMEM((CHUNK, LANES), F32),
        ],
        compiler_params=_SC_PARAMS,
    )
    def deg_kernel(row_hbm, col_hbm, w_hbm, z_hbm, deg_out, wz_out,
                   acc_sh, rowv, colv, wv, wzv, valv):
        c = lax.axis_index("c")
        s = lax.axis_index("s")
        wid = c * NS + s
        pltpu.sync_copy(z_hbm, acc_sh.at[pl.ds(s * stripe, stripe)])
        plsc.subcore_barrier()
        base0 = wid * per_tile
        iota = lax.broadcasted_iota(I32, (LANES,), 0)

        def chunk(i, carry):
            base = base0 + i * CHUNK
            pltpu.sync_copy(row_hbm.at[pl.ds(base, CHUNK)], rowv)
            pltpu.sync_copy(col_hbm.at[pl.ds(base, CHUNK)], colv)
            pltpu.sync_copy(w_hbm.at[pl.ds(base, CHUNK)], wv)
            for j in range(CHUNK // LANES):
                sl = pl.ds(j * LANES, LANES)
                wz = jnp.where(rowv[sl] == colv[sl], 0.0, wv[sl])
                wzv[sl] = wz
                for e in range(LANES):
                    valv[j * LANES + e] = jnp.where(iota == e, wz, 0.0)
            pltpu.sync_copy(wzv, wz_out.at[pl.ds(base, CHUNK)])
            pltpu.sync_copy(valv, acc_sh.at[rowv], add=True)
            return carry

        lax.fori_loop(0, n_chunks, chunk, 0)
        plsc.subcore_barrier()
        pltpu.sync_copy(acc_sh.at[pl.ds(s * stripe, stripe)],
                        deg_out.at[pl.ds(c * n_pad + s * stripe, stripe)])

    return deg_kernel(row_p, col_p, w_p, jnp.zeros((stripe, LANES), F32))


def _sc_edge(row_p, col_p, wz_p, gp):
    e_pad = row_p.shape[0]
    n_pad, width = gp.shape
    per_tile = e_pad // NW
    n_chunks = per_tile // CHUNK
    stripe = n_pad // NS

    @functools.partial(
        pl.kernel,
        out_type=jax.ShapeDtypeStruct((NC * n_pad, width), F32),
        mesh=_mesh(),
        scratch_types=[
            pltpu.VMEM_SHARED((n_pad, width), F32),
            pltpu.VMEM((CHUNK,), I32),
            pltpu.VMEM((CHUNK,), I32),
            pltpu.VMEM((CHUNK,), F32),
            pltpu.VMEM((CHUNK, width), F32),
            pltpu.SemaphoreType.DMA,
        ],
        compiler_params=_SC_PARAMS,
    )
    def edge_kernel(row_hbm, col_hbm, wz_hbm, gp_hbm, z_hbm, acc_out,
                    acc_sh, rowv, colv, wzv, rows_v, sem):
        c = lax.axis_index("c")
        s = lax.axis_index("s")
        wid = c * NS + s
        pltpu.sync_copy(z_hbm, acc_sh.at[pl.ds(s * stripe, stripe)])
        plsc.subcore_barrier()
        base0 = wid * per_tile

        def chunk(i, carry):
            base = base0 + i * CHUNK
            pltpu.sync_copy(row_hbm.at[pl.ds(base, CHUNK)], rowv)
            pltpu.sync_copy(col_hbm.at[pl.ds(base, CHUNK)], colv)
            pltpu.sync_copy(wz_hbm.at[pl.ds(base, CHUNK)], wzv)
            pltpu.async_copy(gp_hbm.at[rowv], rows_v, sem).wait()
            for j in range(CHUNK // LANES):
                wvec = wzv[pl.ds(j * LANES, LANES)]
                for e in range(LANES):
                    ee = j * LANES + e
                    rows_v[ee] = rows_v[ee] * _splat(wvec, e)
            pltpu.sync_copy(rows_v, acc_sh.at[colv], add=True)
            return carry

        lax.fori_loop(0, n_chunks, chunk, 0)
        plsc.subcore_barrier()
        pltpu.sync_copy(acc_sh.at[pl.ds(s * stripe, stripe)],
                        acc_out.at[pl.ds(c * n_pad + s * stripe, stripe)])

    return edge_kernel(row_p, col_p, wz_p, gp,
                       jnp.zeros((stripe, width), F32))


_DOT = functools.partial(
    lax.dot_general,
    precision=lax.Precision.HIGHEST,
    preferred_element_type=F32,
)
_DN = (((1,), (1,)), ((), ()))


def _tc_mm2(h, Wa, Wb, blk=1024):
    n_pad, f = h.shape
    w = Wa.shape[0]

    def body(h_ref, wa_ref, wb_ref, oa_ref, ob_ref):
        hb = h_ref[...]
        oa_ref[...] = _DOT(hb, wa_ref[...], _DN)
        ob_ref[...] = _DOT(hb, wb_ref[...], _DN)

    return pl.pallas_call(
        body,
        grid=(n_pad // blk,),
        in_specs=[
            pl.BlockSpec((blk, f), lambda i: (i, 0)),
            pl.BlockSpec((w, f), lambda i: (0, 0)),
            pl.BlockSpec((w, f), lambda i: (0, 0)),
        ],
        out_specs=[
            pl.BlockSpec((blk, w), lambda i: (i, 0)),
            pl.BlockSpec((blk, w), lambda i: (i, 0)),
        ],
        out_shape=[
            jax.ShapeDtypeStruct((n_pad, w), F32),
            jax.ShapeDtypeStruct((n_pad, w), F32),
        ],
    )(h, Wa, Wb)


def _tc_prep(deg_parts, g1b, blk=1024):
    nc, n_pad, lanes = deg_parts.shape
    width = g1b.shape[1]

    def body(d_ref, g_ref, dinv_ref, gp_ref):
        deg = jnp.sum(d_ref[...], axis=(0, 2), keepdims=False)[:, None]
        pos = deg > 0.0
        dinv = jnp.where(pos, lax.rsqrt(jnp.where(pos, deg, 1.0)), 0.0)
        dinv_bc = jnp.broadcast_to(dinv, (blk, width))
        dinv_ref[...] = dinv_bc
        gp_ref[...] = dinv_bc * g_ref[...]

    return pl.pallas_call(
        body,
        grid=(n_pad // blk,),
        in_specs=[
            pl.BlockSpec((nc, blk, lanes), lambda i: (0, i, 0)),
            pl.BlockSpec((blk, width), lambda i: (i, 0)),
        ],
        out_specs=[
            pl.BlockSpec((blk, width), lambda i: (i, 0)),
            pl.BlockSpec((blk, width), lambda i: (i, 0)),
        ],
        out_shape=[
            jax.ShapeDtypeStruct((n_pad, width), F32),
            jax.ShapeDtypeStruct((n_pad, width), F32),
        ],
    )(deg_parts, g1b)


def _tc_fuse_mid(g1a, s1_parts, dinv_bc, b1, W2a, W2b, blk=1024):
    n_pad, hid = g1a.shape
    w2 = W2a.shape[0]

    def body(ga_ref, s_ref, dinv_ref, b_ref, wa_ref, wb_ref, oa_ref, ogp_ref):
        dinv = dinv_ref[...]
        h = ga_ref[...] - dinv * jnp.sum(s_ref[...], axis=0) + b_ref[...]
        h = jnp.maximum(h, 0.0)
        oa_ref[...] = _DOT(h, wa_ref[...], _DN)
        ogp_ref[...] = dinv * _DOT(h, wb_ref[...], _DN)

    return pl.pallas_call(
        body,
        grid=(n_pad // blk,),
        in_specs=[
            pl.BlockSpec((blk, hid), lambda i: (i, 0)),
            pl.BlockSpec((NC, blk, hid), lambda i: (0, i, 0)),
            pl.BlockSpec((blk, hid), lambda i: (i, 0)),
            pl.BlockSpec((1, hid), lambda i: (0, 0)),
            pl.BlockSpec((w2, hid), lambda i: (0, 0)),
            pl.BlockSpec((w2, hid), lambda i: (0, 0)),
        ],
        out_specs=[
            pl.BlockSpec((blk, w2), lambda i: (i, 0)),
            pl.BlockSpec((blk, w2), lambda i: (i, 0)),
        ],
        out_shape=[
            jax.ShapeDtypeStruct((n_pad, w2), F32),
            jax.ShapeDtypeStruct((n_pad, w2), F32),
        ],
    )(g1a, s1_parts, dinv_bc, b1, W2a, W2b)


def _tc_fuse_out(g2a, s2_parts, dinv_bc, b2, blk=1024):
    n_pad, ncls = g2a.shape

    def body(ga_ref, s_ref, dinv_ref, b_ref, o_ref):
        z = (ga_ref[...] - dinv_ref[...] * jnp.sum(s_ref[...], axis=0)
             + b_ref[...])
        m = jnp.max(z, axis=1, keepdims=True)
        zm = z - m
        o_ref[...] = zm - jnp.log(jnp.sum(jnp.exp(zm), axis=1, keepdims=True))

    return pl.pallas_call(
        body,
        grid=(n_pad // blk,),
        in_specs=[
            pl.BlockSpec((blk, ncls), lambda i: (i, 0)),
            pl.BlockSpec((NC, blk, ncls), lambda i: (0, i, 0)),
            pl.BlockSpec((blk, ncls), lambda i: (i, 0)),
            pl.BlockSpec((1, ncls), lambda i: (0, 0)),
        ],
        out_specs=pl.BlockSpec((blk, ncls), lambda i: (i, 0)),
        out_shape=jax.ShapeDtypeStruct((n_pad, ncls), F32),
    )(g2a, s2_parts, dinv_bc, b2)


def kernel(x, edge_index, edge_attr, W1_0, W1_1, b1, W2_0, W2_1, b2):
    n, f_in = x.shape
    e = edge_attr.shape[0]
    hid = W1_0.shape[0]
    ncls = W2_0.shape[0]

    n_pad = _round_up(n, NS * 128)
    e_pad = _round_up(e, NW * CHUNK)

    row_p = jnp.pad(edge_index[0], (0, e_pad - e))
    col_p = jnp.pad(edge_index[1], (0, e_pad - e))
    w_p = jnp.pad(edge_attr, (0, e_pad - e))
    x_pad = jnp.pad(x, ((0, n_pad - n), (0, 0)))

    deg_parts, wz_p = _sc_deg(row_p, col_p, w_p, n_pad)
    g1a, g1b = _tc_mm2(x_pad, W1_0, W1_1)
    dinv_bc, gp1 = _tc_prep(deg_parts.reshape(NC, n_pad, LANES), g1b)

    s1_flat = _sc_edge(row_p, col_p, wz_p, gp1)
    g2a, gp2 = _tc_fuse_mid(g1a, s1_flat.reshape(NC, n_pad, hid), dinv_bc,
                            b1.reshape(1, hid), W2_0, W2_1)
    s2_flat = _sc_edge(row_p, col_p, wz_p, gp2)
    out = _tc_fuse_out(g2a, s2_flat.reshape(NC, n_pad, ncls), dinv_bc,
                       b2.reshape(1, ncls))
    return out[:n]


# speedup vs baseline: 20.1771x; 1.5757x over previous
import functools

import jax
import jax.numpy as jnp
from jax import lax
from jax.experimental import pallas as pl
from jax.experimental.pallas import tpu as pltpu
from jax.experimental.pallas import tpu_sc as plsc

NC = 2
NS = 16
NW = NC * NS
LANES = 16
CHUNK = 128

F32 = jnp.float32
I32 = jnp.int32

_SC_PARAMS = pltpu.CompilerParams(use_tc_tiling_on_sc=False)


def _round_up(v, m):
    return (v + m - 1) // m * m


def _mesh():
    return plsc.VectorSubcoreMesh(core_axis_name="c", subcore_axis_name="s",
                                  num_cores=NC, num_subcores=NS)


def _splat(vec, e):
    idx = jnp.full((LANES,), e, I32)
    return lax.gather(
        vec, idx[:, None],
        lax.GatherDimensionNumbers(offset_dims=(), collapsed_slice_dims=(0,),
                                   start_index_map=(0,)),
        (1,), mode=lax.GatherScatterMode.PROMISE_IN_BOUNDS)


def _sc_deg(row_p, col_p, w_p, n_pad):
    e_pad = row_p.shape[0]
    per_tile = e_pad // NW
    n_chunks = per_tile // CHUNK
    stripe = n_pad // NS

    @functools.partial(
        pl.kernel,
        out_type=(jax.ShapeDtypeStruct((NC * n_pad, LANES), F32),
                  jax.ShapeDtypeStruct((e_pad,), F32)),
        mesh=_mesh(),
        scratch_types=[
            pltpu.VMEM_SHARED((n_pad, LANES), F32),
            pltpu.VMEM((CHUNK,), I32),
            pltpu.VMEM((CHUNK,), I32),
            pltpu.VMEM((CHUNK,), F32),
            pltpu.VMEM((CHUNK,), F32),
            pltpu.VMEM((CHUNK, LANES), F32),
        ],
        compiler_params=_SC_PARAMS,
    )
    def deg_kernel(row_hbm, col_hbm, w_hbm, z_hbm, deg_out, wz_out,
                   acc_sh, rowv, colv, wv, wzv, valv):
        c = lax.axis_index("c")
        s = lax.axis_index("s")
        wid = c * NS + s
        pltpu.sync_copy(z_hbm, acc_sh.at[pl.ds(s * stripe, stripe)])
        plsc.subcore_barrier()
        base0 = wid * per_tile
        iota = lax.broadcasted_iota(I32, (LANES,), 0)

        def chunk(i, carry):
            base = base0 + i * CHUNK
            pltpu.sync_copy(row_hbm.at[pl.ds(base, CHUNK)], rowv)
            pltpu.sync_copy(col_hbm.at[pl.ds(base, CHUNK)], colv)
            pltpu.sync_copy(w_hbm.at[pl.ds(base, CHUNK)], wv)
            for j in range(CHUNK // LANES):
                sl = pl.ds(j * LANES, LANES)
                wz = jnp.where(rowv[sl] == colv[sl], 0.0, wv[sl])
                wzv[sl] = wz
                for e in range(LANES):
                    valv[j * LANES + e] = jnp.where(iota == e, wz, 0.0)
            pltpu.sync_copy(wzv, wz_out.at[pl.ds(base, CHUNK)])
            pltpu.sync_copy(valv, acc_sh.at[rowv], add=True)
            return carry

        lax.fori_loop(0, n_chunks, chunk, 0)
        plsc.subcore_barrier()
        pltpu.sync_copy(acc_sh.at[pl.ds(s * stripe, stripe)],
                        deg_out.at[pl.ds(c * n_pad + s * stripe, stripe)])

    return deg_kernel(row_p, col_p, w_p, jnp.zeros((stripe, LANES), F32))


SUPER = 4
SEDGES = SUPER * CHUNK


def _sc_edge(row2, col2, wz2, gp):
    t_rows = row2.shape[0]
    e_pad = t_rows * CHUNK
    n_pad, width = gp.shape
    per_tile = e_pad // NW
    n_steps = per_tile // SEDGES
    assert n_steps % 2 == 0 and n_steps >= 4
    stripe = n_pad // NS

    @functools.partial(
        pl.kernel,
        out_type=jax.ShapeDtypeStruct((NC * n_pad, width), F32),
        mesh=_mesh(),
        scratch_types=[
            pltpu.VMEM_SHARED((n_pad, width), F32),
            pltpu.VMEM((SUPER, CHUNK), I32), pltpu.VMEM((SUPER, CHUNK), I32),
            pltpu.VMEM((SUPER, CHUNK), I32), pltpu.VMEM((SUPER, CHUNK), I32),
            pltpu.VMEM((SUPER, CHUNK), F32), pltpu.VMEM((SUPER, CHUNK), F32),
            pltpu.VMEM((SEDGES, width), F32), pltpu.VMEM((SEDGES, width), F32),
            pltpu.SemaphoreType.DMA, pltpu.SemaphoreType.DMA,
        ],
        compiler_params=_SC_PARAMS,
    )
    def edge_kernel(row_hbm, col_hbm, wz_hbm, gp_hbm, z_hbm, acc_out,
                    acc_sh, rowv0, rowv1, colv0, colv1, wzv0, wzv1,
                    rows0, rows1, sem_l, sem_g):
        c = lax.axis_index("c")
        s = lax.axis_index("s")
        wid = c * NS + s
        pltpu.sync_copy(z_hbm, acc_sh.at[pl.ds(s * stripe, stripe)])
        plsc.subcore_barrier()
        rowv = (rowv0, rowv1)
        colv = (colv0, colv1)
        wzv = (wzv0, wzv1)
        rows = (rows0, rows1)
        base0 = wid * (per_tile // CHUNK)

        def issue_loads(u, p):
            sl = pl.ds(base0 + u * SUPER, SUPER)
            pltpu.async_copy(row_hbm.at[sl], rowv[p], sem_l)
            pltpu.async_copy(col_hbm.at[sl], colv[p], sem_l)
            pltpu.async_copy(wz_hbm.at[sl], wzv[p], sem_l)

        def wait_loads(u, p):
            sl = pl.ds(base0 + u * SUPER, SUPER)
            pltpu.make_async_copy(row_hbm.at[sl], rowv[p], sem_l).wait()
            pltpu.make_async_copy(col_hbm.at[sl], colv[p], sem_l).wait()
            pltpu.make_async_copy(wz_hbm.at[sl], wzv[p], sem_l).wait()

        def fire_gathers(p):
            for k in range(SUPER):
                pltpu.async_copy(gp_hbm.at[rowv[p].at[k]],
                                 rows[p].at[pl.ds(k * CHUNK, CHUNK)], sem_g)

        def wait_gathers(p):
            for k in range(SUPER):
                pltpu.make_async_copy(
                    gp_hbm.at[rowv[p].at[k]],
                    rows[p].at[pl.ds(k * CHUNK, CHUNK)], sem_g).wait()

        def compute_scatter(p):
            rv = rows[p]
            for k in range(SUPER):
                for j in range(CHUNK // LANES):
                    wvec = wzv[p][k, pl.ds(j * LANES, LANES)]
                    for e in range(LANES):
                        ee = k * CHUNK + j * LANES + e
                        rv[ee] = rv[ee] * _splat(wvec, e)
                pltpu.sync_copy(rv.at[pl.ds(k * CHUNK, CHUNK)],
                                acc_sh.at[colv[p].at[k]], add=True)

        issue_loads(0, 0)
        wait_loads(0, 0)
        fire_gathers(0)
        issue_loads(1, 1)

        def step(u, p):
            @pl.when(u + 1 < n_steps)
            def _():
                wait_loads(u + 1, 1 - p)
                fire_gathers(1 - p)
            wait_gathers(p)
            compute_scatter(p)
            @pl.when(u + 2 < n_steps)
            def _():
                issue_loads(u + 2, p)

        def round_(r, carry):
            step(2 * r, 0)
            step(2 * r + 1, 1)
            return carry

        lax.fori_loop(0, n_steps // 2, round_, 0)
        plsc.subcore_barrier()
        pltpu.sync_copy(acc_sh.at[pl.ds(s * stripe, stripe)],
                        acc_out.at[pl.ds(c * n_pad + s * stripe, stripe)])

    return edge_kernel(row2, col2, wz2, gp,
                       jnp.zeros((stripe, width), F32))


_DOT = functools.partial(
    lax.dot_general,
    precision=lax.Precision.HIGHEST,
    preferred_element_type=F32,
)
_DN = (((1,), (1,)), ((), ()))


def _tc_mm2(h, Wa, Wb, blk=1024):
    n_pad, f = h.shape
    w = Wa.shape[0]

    def body(h_ref, wa_ref, wb_ref, oa_ref, ob_ref):
        hb = h_ref[...]
        oa_ref[...] = _DOT(hb, wa_ref[...], _DN)
        ob_ref[...] = _DOT(hb, wb_ref[...], _DN)

    return pl.pallas_call(
        body,
        grid=(n_pad // blk,),
        in_specs=[
            pl.BlockSpec((blk, f), lambda i: (i, 0)),
            pl.BlockSpec((w, f), lambda i: (0, 0)),
            pl.BlockSpec((w, f), lambda i: (0, 0)),
        ],
        out_specs=[
            pl.BlockSpec((blk, w), lambda i: (i, 0)),
            pl.BlockSpec((blk, w), lambda i: (i, 0)),
        ],
        out_shape=[
            jax.ShapeDtypeStruct((n_pad, w), F32),
            jax.ShapeDtypeStruct((n_pad, w), F32),
        ],
    )(h, Wa, Wb)


def _tc_prep(deg_parts, g1b, blk=1024):
    nc, n_pad, lanes = deg_parts.shape
    width = g1b.shape[1]

    def body(d_ref, g_ref, dinv_ref, gp_ref):
        deg = jnp.sum(d_ref[...], axis=(0, 2), keepdims=False)[:, None]
        pos = deg > 0.0
        dinv = jnp.where(pos, lax.rsqrt(jnp.where(pos, deg, 1.0)), 0.0)
        dinv_bc = jnp.broadcast_to(dinv, (blk, width))
        dinv_ref[...] = dinv_bc
        gp_ref[...] = dinv_bc * g_ref[...]

    return pl.pallas_call(
        body,
        grid=(n_pad // blk,),
        in_specs=[
            pl.BlockSpec((nc, blk, lanes), lambda i: (0, i, 0)),
            pl.BlockSpec((blk, width), lambda i: (i, 0)),
        ],
        out_specs=[
            pl.BlockSpec((blk, width), lambda i: (i, 0)),
            pl.BlockSpec((blk, width), lambda i: (i, 0)),
        ],
        out_shape=[
            jax.ShapeDtypeStruct((n_pad, width), F32),
            jax.ShapeDtypeStruct((n_pad, width), F32),
        ],
    )(deg_parts, g1b)


def _tc_fuse_mid(g1a, s1_parts, dinv_bc, b1, W2a, W2b, blk=1024):
    n_pad, hid = g1a.shape
    w2 = W2a.shape[0]

    def body(ga_ref, s_ref, dinv_ref, b_ref, wa_ref, wb_ref, oa_ref, ogp_ref):
        dinv = dinv_ref[...]
        h = ga_ref[...] - dinv * jnp.sum(s_ref[...], axis=0) + b_ref[...]
        h = jnp.maximum(h, 0.0)
        oa_ref[...] = _DOT(h, wa_ref[...], _DN)
        ogp_ref[...] = dinv * _DOT(h, wb_ref[...], _DN)

    return pl.pallas_call(
        body,
        grid=(n_pad // blk,),
        in_specs=[
            pl.BlockSpec((blk, hid), lambda i: (i, 0)),
            pl.BlockSpec((NC, blk, hid), lambda i: (0, i, 0)),
            pl.BlockSpec((blk, hid), lambda i: (i, 0)),
            pl.BlockSpec((1, hid), lambda i: (0, 0)),
            pl.BlockSpec((w2, hid), lambda i: (0, 0)),
            pl.BlockSpec((w2, hid), lambda i: (0, 0)),
        ],
        out_specs=[
            pl.BlockSpec((blk, w2), lambda i: (i, 0)),
            pl.BlockSpec((blk, w2), lambda i: (i, 0)),
        ],
        out_shape=[
            jax.ShapeDtypeStruct((n_pad, w2), F32),
            jax.ShapeDtypeStruct((n_pad, w2), F32),
        ],
    )(g1a, s1_parts, dinv_bc, b1, W2a, W2b)


def _tc_fuse_out(g2a, s2_parts, dinv_bc, b2, blk=1024):
    n_pad, ncls = g2a.shape

    def body(ga_ref, s_ref, dinv_ref, b_ref, o_ref):
        z = (ga_ref[...] - dinv_ref[...] * jnp.sum(s_ref[...], axis=0)
             + b_ref[...])
        m = jnp.max(z, axis=1, keepdims=True)
        zm = z - m
        o_ref[...] = zm - jnp.log(jnp.sum(jnp.exp(zm), axis=1, keepdims=True))

    return pl.pallas_call(
        body,
        grid=(n_pad // blk,),
        in_specs=[
            pl.BlockSpec((blk, ncls), lambda i: (i, 0)),
            pl.BlockSpec((NC, blk, ncls), lambda i: (0, i, 0)),
            pl.BlockSpec((blk, ncls), lambda i: (i, 0)),
            pl.BlockSpec((1, ncls), lambda i: (0, 0)),
        ],
        out_specs=pl.BlockSpec((blk, ncls), lambda i: (i, 0)),
        out_shape=jax.ShapeDtypeStruct((n_pad, ncls), F32),
    )(g2a, s2_parts, dinv_bc, b2)


def kernel(x, edge_index, edge_attr, W1_0, W1_1, b1, W2_0, W2_1, b2):
    n, f_in = x.shape
    e = edge_attr.shape[0]
    hid = W1_0.shape[0]
    ncls = W2_0.shape[0]

    n_pad = _round_up(n, NS * 128)
    e_pad = _round_up(e, NW * SEDGES * 2)

    row_p = jnp.pad(edge_index[0], (0, e_pad - e))
    col_p = jnp.pad(edge_index[1], (0, e_pad - e))
    w_p = jnp.pad(edge_attr, (0, e_pad - e))
    x_pad = jnp.pad(x, ((0, n_pad - n), (0, 0)))

    deg_parts, wz_p = _sc_deg(row_p, col_p, w_p, n_pad)
    g1a, g1b = _tc_mm2(x_pad, W1_0, W1_1)
    dinv_bc, gp1 = _tc_prep(deg_parts.reshape(NC, n_pad, LANES), g1b)

    row2 = row_p.reshape(-1, CHUNK)
    col2 = col_p.reshape(-1, CHUNK)
    wz2 = wz_p.reshape(-1, CHUNK)
    s1_flat = _sc_edge(row2, col2, wz2, gp1)
    g2a, gp2 = _tc_fuse_mid(g1a, s1_flat.reshape(NC, n_pad, hid), dinv_bc,
                            b1.reshape(1, hid), W2_0, W2_1)
    s2_flat = _sc_edge(row2, col2, wz2, gp2)
    out = _tc_fuse_out(g2a, s2_flat.reshape(NC, n_pad, ncls), dinv_bc,
                       b2.reshape(1, ncls))
    return out[:n]


# speedup vs baseline: 28.2065x; 1.3979x over previous
import functools

import jax
import jax.numpy as jnp
from jax import lax
from jax.experimental import pallas as pl
from jax.experimental.pallas import tpu as pltpu
from jax.experimental.pallas import tpu_sc as plsc

NC = 2
NS = 16
NW = NC * NS
LANES = 16
CHUNK = 128

F32 = jnp.float32
I32 = jnp.int32

_SC_PARAMS = pltpu.CompilerParams(use_tc_tiling_on_sc=False)


def _round_up(v, m):
    return (v + m - 1) // m * m


def _mesh():
    return plsc.VectorSubcoreMesh(core_axis_name="c", subcore_axis_name="s",
                                  num_cores=NC, num_subcores=NS)


def _splat(vec, e):
    idx = jnp.full((LANES,), e, I32)
    return lax.gather(
        vec, idx[:, None],
        lax.GatherDimensionNumbers(offset_dims=(), collapsed_slice_dims=(0,),
                                   start_index_map=(0,)),
        (1,), mode=lax.GatherScatterMode.PROMISE_IN_BOUNDS)


def _sc_deg(row2, col2, w2, n_pad):
    t_rows = row2.shape[0]
    e_pad = t_rows * CHUNK
    per_tile = e_pad // NW
    n_steps = per_tile // SEDGES
    assert n_steps % 2 == 0 and n_steps >= 4
    stripe = n_pad // NS

    @functools.partial(
        pl.kernel,
        out_type=(jax.ShapeDtypeStruct((NC * n_pad, LANES), F32),
                  jax.ShapeDtypeStruct((t_rows, CHUNK), F32)),
        mesh=_mesh(),
        scratch_types=[
            pltpu.VMEM_SHARED((n_pad, LANES), F32),
            pltpu.VMEM((SUPER, CHUNK), I32), pltpu.VMEM((SUPER, CHUNK), I32),
            pltpu.VMEM((SUPER, CHUNK), I32), pltpu.VMEM((SUPER, CHUNK), I32),
            pltpu.VMEM((SUPER, CHUNK), F32), pltpu.VMEM((SUPER, CHUNK), F32),
            pltpu.VMEM((SUPER, CHUNK), F32), pltpu.VMEM((SUPER, CHUNK), F32),
            pltpu.VMEM((SEDGES, LANES), F32),
            pltpu.SemaphoreType.DMA, pltpu.SemaphoreType.DMA,
        ],
        compiler_params=_SC_PARAMS,
    )
    def deg_kernel(row_hbm, col_hbm, w_hbm, z_hbm, deg_out, wz_out,
                   acc_sh, rowv0, rowv1, colv0, colv1, wv0, wv1,
                   wzv0, wzv1, valv, sem_l, sem_w):
        c = lax.axis_index("c")
        s = lax.axis_index("s")
        wid = c * NS + s
        pltpu.sync_copy(z_hbm, acc_sh.at[pl.ds(s * stripe, stripe)])
        plsc.subcore_barrier()
        rowv = (rowv0, rowv1)
        colv = (colv0, colv1)
        wv = (wv0, wv1)
        wzv = (wzv0, wzv1)
        base0 = wid * (per_tile // CHUNK)
        iota = lax.broadcasted_iota(I32, (LANES,), 0)

        def issue_loads(u, p):
            sl = pl.ds(base0 + u * SUPER, SUPER)
            pltpu.async_copy(row_hbm.at[sl], rowv[p], sem_l)
            pltpu.async_copy(col_hbm.at[sl], colv[p], sem_l)
            pltpu.async_copy(w_hbm.at[sl], wv[p], sem_l)

        def wait_loads(u, p):
            sl = pl.ds(base0 + u * SUPER, SUPER)
            pltpu.make_async_copy(row_hbm.at[sl], rowv[p], sem_l).wait()
            pltpu.make_async_copy(col_hbm.at[sl], colv[p], sem_l).wait()
            pltpu.make_async_copy(w_hbm.at[sl], wv[p], sem_l).wait()

        def wz_slice(u):
            return pl.ds(base0 + u * SUPER, SUPER)

        issue_loads(0, 0)
        wait_loads(0, 0)
        issue_loads(1, 1)

        def step(u, p):
            @pl.when(u >= 2)
            def _():
                pltpu.make_async_copy(wzv[p], wz_out.at[wz_slice(u - 2)],
                                      sem_w).wait()
            for k in range(SUPER):
                for j in range(CHUNK // LANES):
                    sl = pl.ds(j * LANES, LANES)
                    wz = jnp.where(rowv[p][k, sl] == colv[p][k, sl],
                                   0.0, wv[p][k, sl])
                    wzv[p][k, sl] = wz
                    for e in range(LANES):
                        valv[k * CHUNK + j * LANES + e] = (
                            jnp.where(iota == e, wz, 0.0))
                pltpu.sync_copy(valv.at[pl.ds(k * CHUNK, CHUNK)],
                                acc_sh.at[rowv[p].at[k]], add=True)
            pltpu.async_copy(wzv[p], wz_out.at[wz_slice(u)], sem_w)
            @pl.when(u + 1 < n_steps)
            def _():
                wait_loads(u + 1, 1 - p)
            @pl.when(u + 2 < n_steps)
            def _():
                issue_loads(u + 2, p)

        def round_(r, carry):
            step(2 * r, 0)
            step(2 * r + 1, 1)
            return carry

        lax.fori_loop(0, n_steps // 2, round_, 0)
        pltpu.make_async_copy(wzv[0], wz_out.at[wz_slice(n_steps - 2)],
                              sem_w).wait()
        pltpu.make_async_copy(wzv[1], wz_out.at[wz_slice(n_steps - 1)],
                              sem_w).wait()
        plsc.subcore_barrier()
        pltpu.sync_copy(acc_sh.at[pl.ds(s * stripe, stripe)],
                        deg_out.at[pl.ds(c * n_pad + s * stripe, stripe)])

    return deg_kernel(row2, col2, w2, jnp.zeros((stripe, LANES), F32))


SUPER = 4
SEDGES = SUPER * CHUNK


def _sc_edge(row2, col2, wz2, gp):
    t_rows = row2.shape[0]
    e_pad = t_rows * CHUNK
    n_pad, width = gp.shape
    per_tile = e_pad // NW
    n_steps = per_tile // SEDGES
    assert n_steps % 2 == 0 and n_steps >= 4
    stripe = n_pad // NS

    @functools.partial(
        pl.kernel,
        out_type=jax.ShapeDtypeStruct((NC * n_pad, width), F32),
        mesh=_mesh(),
        scratch_types=[
            pltpu.VMEM_SHARED((n_pad, width), F32),
            pltpu.VMEM((SUPER, CHUNK), I32), pltpu.VMEM((SUPER, CHUNK), I32),
            pltpu.VMEM((SUPER, CHUNK), I32), pltpu.VMEM((SUPER, CHUNK), I32),
            pltpu.VMEM((SUPER, CHUNK), F32), pltpu.VMEM((SUPER, CHUNK), F32),
            pltpu.VMEM((SEDGES, width), F32), pltpu.VMEM((SEDGES, width), F32),
            pltpu.SemaphoreType.DMA, pltpu.SemaphoreType.DMA,
        ],
        compiler_params=_SC_PARAMS,
    )
    def edge_kernel(row_hbm, col_hbm, wz_hbm, gp_hbm, z_hbm, acc_out,
                    acc_sh, rowv0, rowv1, colv0, colv1, wzv0, wzv1,
                    rows0, rows1, sem_l, sem_g):
        c = lax.axis_index("c")
        s = lax.axis_index("s")
        wid = c * NS + s
        pltpu.sync_copy(z_hbm, acc_sh.at[pl.ds(s * stripe, stripe)])
        plsc.subcore_barrier()
        rowv = (rowv0, rowv1)
        colv = (colv0, colv1)
        wzv = (wzv0, wzv1)
        rows = (rows0, rows1)
        base0 = wid * (per_tile // CHUNK)

        def issue_loads(u, p):
            sl = pl.ds(base0 + u * SUPER, SUPER)
            pltpu.async_copy(row_hbm.at[sl], rowv[p], sem_l)
            pltpu.async_copy(col_hbm.at[sl], colv[p], sem_l)
            pltpu.async_copy(wz_hbm.at[sl], wzv[p], sem_l)

        def wait_loads(u, p):
            sl = pl.ds(base0 + u * SUPER, SUPER)
            pltpu.make_async_copy(row_hbm.at[sl], rowv[p], sem_l).wait()
            pltpu.make_async_copy(col_hbm.at[sl], colv[p], sem_l).wait()
            pltpu.make_async_copy(wz_hbm.at[sl], wzv[p], sem_l).wait()

        def fire_gathers(p):
            for k in range(SUPER):
                pltpu.async_copy(gp_hbm.at[rowv[p].at[k]],
                                 rows[p].at[pl.ds(k * CHUNK, CHUNK)], sem_g)

        def wait_gathers(p):
            for k in range(SUPER):
                pltpu.make_async_copy(
                    gp_hbm.at[rowv[p].at[k]],
                    rows[p].at[pl.ds(k * CHUNK, CHUNK)], sem_g).wait()

        def compute_scatter(p):
            rv = rows[p]
            for k in range(SUPER):
                for j in range(CHUNK // LANES):
                    wvec = wzv[p][k, pl.ds(j * LANES, LANES)]
                    for e in range(LANES):
                        ee = k * CHUNK + j * LANES + e
                        rv[ee] = rv[ee] * _splat(wvec, e)
                pltpu.sync_copy(rv.at[pl.ds(k * CHUNK, CHUNK)],
                                acc_sh.at[colv[p].at[k]], add=True)

        issue_loads(0, 0)
        wait_loads(0, 0)
        fire_gathers(0)
        issue_loads(1, 1)

        def step(u, p):
            @pl.when(u + 1 < n_steps)
            def _():
                wait_loads(u + 1, 1 - p)
                fire_gathers(1 - p)
            wait_gathers(p)
            compute_scatter(p)
            @pl.when(u + 2 < n_steps)
            def _():
                issue_loads(u + 2, p)

        def round_(r, carry):
            step(2 * r, 0)
            step(2 * r + 1, 1)
            return carry

        lax.fori_loop(0, n_steps // 2, round_, 0)
        plsc.subcore_barrier()
        pltpu.sync_copy(acc_sh.at[pl.ds(s * stripe, stripe)],
                        acc_out.at[pl.ds(c * n_pad + s * stripe, stripe)])

    return edge_kernel(row2, col2, wz2, gp,
                       jnp.zeros((stripe, width), F32))


_DOT = functools.partial(
    lax.dot_general,
    precision=lax.Precision.HIGHEST,
    preferred_element_type=F32,
)
_DN = (((1,), (1,)), ((), ()))


def _tc_mm2(h, Wa, Wb, blk=1024):
    n_pad, f = h.shape
    w = Wa.shape[0]

    def body(h_ref, wa_ref, wb_ref, oa_ref, ob_ref):
        hb = h_ref[...]
        oa_ref[...] = _DOT(hb, wa_ref[...], _DN)
        ob_ref[...] = _DOT(hb, wb_ref[...], _DN)

    return pl.pallas_call(
        body,
        grid=(n_pad // blk,),
        in_specs=[
            pl.BlockSpec((blk, f), lambda i: (i, 0)),
            pl.BlockSpec((w, f), lambda i: (0, 0)),
            pl.BlockSpec((w, f), lambda i: (0, 0)),
        ],
        out_specs=[
            pl.BlockSpec((blk, w), lambda i: (i, 0)),
            pl.BlockSpec((blk, w), lambda i: (i, 0)),
        ],
        out_shape=[
            jax.ShapeDtypeStruct((n_pad, w), F32),
            jax.ShapeDtypeStruct((n_pad, w), F32),
        ],
    )(h, Wa, Wb)


def _tc_prep(deg_parts, g1b, blk=1024):
    nc, n_pad, lanes = deg_parts.shape
    width = g1b.shape[1]

    def body(d_ref, g_ref, dinv_ref, gp_ref):
        deg = jnp.sum(d_ref[...], axis=(0, 2), keepdims=False)[:, None]
        pos = deg > 0.0
        dinv = jnp.where(pos, lax.rsqrt(jnp.where(pos, deg, 1.0)), 0.0)
        dinv_bc = jnp.broadcast_to(dinv, (blk, width))
        dinv_ref[...] = dinv_bc
        gp_ref[...] = dinv_bc * g_ref[...]

    return pl.pallas_call(
        body,
        grid=(n_pad // blk,),
        in_specs=[
            pl.BlockSpec((nc, blk, lanes), lambda i: (0, i, 0)),
            pl.BlockSpec((blk, width), lambda i: (i, 0)),
        ],
        out_specs=[
            pl.BlockSpec((blk, width), lambda i: (i, 0)),
            pl.BlockSpec((blk, width), lambda i: (i, 0)),
        ],
        out_shape=[
            jax.ShapeDtypeStruct((n_pad, width), F32),
            jax.ShapeDtypeStruct((n_pad, width), F32),
        ],
    )(deg_parts, g1b)


def _tc_fuse_mid(g1a, s1_parts, dinv_bc, b1, W2a, W2b, blk=1024):
    n_pad, hid = g1a.shape
    w2 = W2a.shape[0]

    def body(ga_ref, s_ref, dinv_ref, b_ref, wa_ref, wb_ref, oa_ref, ogp_ref):
        dinv = dinv_ref[...]
        h = ga_ref[...] - dinv * jnp.sum(s_ref[...], axis=0) + b_ref[...]
        h = jnp.maximum(h, 0.0)
        oa_ref[...] = _DOT(h, wa_ref[...], _DN)
        ogp_ref[...] = dinv * _DOT(h, wb_ref[...], _DN)

    return pl.pallas_call(
        body,
        grid=(n_pad // blk,),
        in_specs=[
            pl.BlockSpec((blk, hid), lambda i: (i, 0)),
            pl.BlockSpec((NC, blk, hid), lambda i: (0, i, 0)),
            pl.BlockSpec((blk, hid), lambda i: (i, 0)),
            pl.BlockSpec((1, hid), lambda i: (0, 0)),
            pl.BlockSpec((w2, hid), lambda i: (0, 0)),
            pl.BlockSpec((w2, hid), lambda i: (0, 0)),
        ],
        out_specs=[
            pl.BlockSpec((blk, w2), lambda i: (i, 0)),
            pl.BlockSpec((blk, w2), lambda i: (i, 0)),
        ],
        out_shape=[
            jax.ShapeDtypeStruct((n_pad, w2), F32),
            jax.ShapeDtypeStruct((n_pad, w2), F32),
        ],
    )(g1a, s1_parts, dinv_bc, b1, W2a, W2b)


def _tc_fuse_out(g2a, s2_parts, dinv_bc, b2, blk=1024):
    n_pad, ncls = g2a.shape

    def body(ga_ref, s_ref, dinv_ref, b_ref, o_ref):
        z = (ga_ref[...] - dinv_ref[...] * jnp.sum(s_ref[...], axis=0)
             + b_ref[...])
        m = jnp.max(z, axis=1, keepdims=True)
        zm = z - m
        o_ref[...] = zm - jnp.log(jnp.sum(jnp.exp(zm), axis=1, keepdims=True))

    return pl.pallas_call(
        body,
        grid=(n_pad // blk,),
        in_specs=[
            pl.BlockSpec((blk, ncls), lambda i: (i, 0)),
            pl.BlockSpec((NC, blk, ncls), lambda i: (0, i, 0)),
            pl.BlockSpec((blk, ncls), lambda i: (i, 0)),
            pl.BlockSpec((1, ncls), lambda i: (0, 0)),
        ],
        out_specs=pl.BlockSpec((blk, ncls), lambda i: (i, 0)),
        out_shape=jax.ShapeDtypeStruct((n_pad, ncls), F32),
    )(g2a, s2_parts, dinv_bc, b2)


def kernel(x, edge_index, edge_attr, W1_0, W1_1, b1, W2_0, W2_1, b2):
    n, f_in = x.shape
    e = edge_attr.shape[0]
    hid = W1_0.shape[0]
    ncls = W2_0.shape[0]

    n_pad = _round_up(n, NS * 128)
    e_pad = _round_up(e, NW * SEDGES * 2)

    row_p = jnp.pad(edge_index[0], (0, e_pad - e))
    col_p = jnp.pad(edge_index[1], (0, e_pad - e))
    w_p = jnp.pad(edge_attr, (0, e_pad - e))
    x_pad = jnp.pad(x, ((0, n_pad - n), (0, 0)))

    row2 = row_p.reshape(-1, CHUNK)
    col2 = col_p.reshape(-1, CHUNK)
    deg_parts, wz2 = _sc_deg(row2, col2, w_p.reshape(-1, CHUNK), n_pad)
    g1a, g1b = _tc_mm2(x_pad, W1_0, W1_1)
    dinv_bc, gp1 = _tc_prep(deg_parts.reshape(NC, n_pad, LANES), g1b)

    s1_flat = _sc_edge(row2, col2, wz2, gp1)
    g2a, gp2 = _tc_fuse_mid(g1a, s1_flat.reshape(NC, n_pad, hid), dinv_bc,
                            b1.reshape(1, hid), W2_0, W2_1)
    s2_flat = _sc_edge(row2, col2, wz2, gp2)
    out = _tc_fuse_out(g2a, s2_flat.reshape(NC, n_pad, ncls), dinv_bc,
                       b2.reshape(1, ncls))
    return out[:n]


# speedup vs baseline: 29.6215x; 1.0502x over previous
import functools

import jax
import jax.numpy as jnp
from jax import lax
from jax.experimental import pallas as pl
from jax.experimental.pallas import tpu as pltpu
from jax.experimental.pallas import tpu_sc as plsc

NC = 2
NS = 16
NW = NC * NS
LANES = 16
CHUNK = 128

F32 = jnp.float32
I32 = jnp.int32

_SC_PARAMS = pltpu.CompilerParams(use_tc_tiling_on_sc=False)


def _round_up(v, m):
    return (v + m - 1) // m * m


def _mesh():
    return plsc.VectorSubcoreMesh(core_axis_name="c", subcore_axis_name="s",
                                  num_cores=NC, num_subcores=NS)


def _splat(vec, e):
    idx = jnp.full((LANES,), e, I32)
    return lax.gather(
        vec, idx[:, None],
        lax.GatherDimensionNumbers(offset_dims=(), collapsed_slice_dims=(0,),
                                   start_index_map=(0,)),
        (1,), mode=lax.GatherScatterMode.PROMISE_IN_BOUNDS)


def _sc_deg(row2, col2, w2, n_pad):
    t_rows = row2.shape[0]
    e_pad = t_rows * CHUNK
    per_tile = e_pad // NW
    n_steps = per_tile // SEDGES
    assert n_steps % 2 == 0 and n_steps >= 4
    stripe = n_pad // NS

    @functools.partial(
        pl.kernel,
        out_type=(jax.ShapeDtypeStruct((NC * n_pad, LANES), F32),
                  jax.ShapeDtypeStruct((t_rows, CHUNK), F32)),
        mesh=_mesh(),
        scratch_types=[
            pltpu.VMEM_SHARED((n_pad, LANES), F32),
            pltpu.VMEM((SUPER, CHUNK), I32), pltpu.VMEM((SUPER, CHUNK), I32),
            pltpu.VMEM((SUPER, CHUNK), I32), pltpu.VMEM((SUPER, CHUNK), I32),
            pltpu.VMEM((SUPER, CHUNK), F32), pltpu.VMEM((SUPER, CHUNK), F32),
            pltpu.VMEM((SUPER, CHUNK), F32), pltpu.VMEM((SUPER, CHUNK), F32),
            pltpu.VMEM((SEDGES, LANES), F32),
            pltpu.SemaphoreType.DMA, pltpu.SemaphoreType.DMA,
        ],
        compiler_params=_SC_PARAMS,
    )
    def deg_kernel(row_hbm, col_hbm, w_hbm, z_hbm, deg_out, wz_out,
                   acc_sh, rowv0, rowv1, colv0, colv1, wv0, wv1,
                   wzv0, wzv1, valv, sem_l, sem_w):
        c = lax.axis_index("c")
        s = lax.axis_index("s")
        wid = c * NS + s
        pltpu.sync_copy(z_hbm, acc_sh.at[pl.ds(s * stripe, stripe)])
        plsc.subcore_barrier()
        rowv = (rowv0, rowv1)
        colv = (colv0, colv1)
        wv = (wv0, wv1)
        wzv = (wzv0, wzv1)
        base0 = wid * (per_tile // CHUNK)
        iota = lax.broadcasted_iota(I32, (LANES,), 0)

        def issue_loads(u, p):
            sl = pl.ds(base0 + u * SUPER, SUPER)
            pltpu.async_copy(row_hbm.at[sl], rowv[p], sem_l)
            pltpu.async_copy(col_hbm.at[sl], colv[p], sem_l)
            pltpu.async_copy(w_hbm.at[sl], wv[p], sem_l)

        def wait_loads(u, p):
            sl = pl.ds(base0 + u * SUPER, SUPER)
            pltpu.make_async_copy(row_hbm.at[sl], rowv[p], sem_l).wait()
            pltpu.make_async_copy(col_hbm.at[sl], colv[p], sem_l).wait()
            pltpu.make_async_copy(w_hbm.at[sl], wv[p], sem_l).wait()

        def wz_slice(u):
            return pl.ds(base0 + u * SUPER, SUPER)

        issue_loads(0, 0)
        wait_loads(0, 0)
        issue_loads(1, 1)

        def step(u, p):
            @pl.when(u >= 2)
            def _():
                pltpu.make_async_copy(wzv[p], wz_out.at[wz_slice(u - 2)],
                                      sem_w).wait()
            for k in range(SUPER):
                for j in range(CHUNK // LANES):
                    sl = pl.ds(j * LANES, LANES)
                    wz = jnp.where(rowv[p][k, sl] == colv[p][k, sl],
                                   0.0, wv[p][k, sl])
                    wzv[p][k, sl] = wz
                    for e in range(LANES):
                        valv[k * CHUNK + j * LANES + e] = (
                            jnp.where(iota == e, wz, 0.0))
                pltpu.sync_copy(valv.at[pl.ds(k * CHUNK, CHUNK)],
                                acc_sh.at[rowv[p].at[k]], add=True)
            pltpu.async_copy(wzv[p], wz_out.at[wz_slice(u)], sem_w)
            @pl.when(u + 1 < n_steps)
            def _():
                wait_loads(u + 1, 1 - p)
            @pl.when(u + 2 < n_steps)
            def _():
                issue_loads(u + 2, p)

        def round_(r, carry):
            step(2 * r, 0)
            step(2 * r + 1, 1)
            return carry

        lax.fori_loop(0, n_steps // 2, round_, 0)
        pltpu.make_async_copy(wzv[0], wz_out.at[wz_slice(n_steps - 2)],
                              sem_w).wait()
        pltpu.make_async_copy(wzv[1], wz_out.at[wz_slice(n_steps - 1)],
                              sem_w).wait()
        plsc.subcore_barrier()
        pltpu.sync_copy(acc_sh.at[pl.ds(s * stripe, stripe)],
                        deg_out.at[pl.ds(c * n_pad + s * stripe, stripe)])

    return deg_kernel(row2, col2, w2, jnp.zeros((stripe, LANES), F32))


SUPER = 4
SEDGES = SUPER * CHUNK


def _sc_edge(row2, col2, wz2, gp):
    t_rows = row2.shape[0]
    e_pad = t_rows * CHUNK
    n_pad, width = gp.shape
    per_tile = e_pad // NW
    n_steps = per_tile // SEDGES
    assert n_steps % 2 == 0 and n_steps >= 4
    stripe = n_pad // NS

    @functools.partial(
        pl.kernel,
        out_type=jax.ShapeDtypeStruct((NC * n_pad, width), F32),
        mesh=_mesh(),
        scratch_types=[
            pltpu.VMEM_SHARED((n_pad, width), F32),
            pltpu.VMEM((SUPER, CHUNK), I32), pltpu.VMEM((SUPER, CHUNK), I32),
            pltpu.VMEM((SUPER, CHUNK), I32), pltpu.VMEM((SUPER, CHUNK), I32),
            pltpu.VMEM((SUPER, CHUNK), F32), pltpu.VMEM((SUPER, CHUNK), F32),
            pltpu.VMEM((SEDGES, width), F32), pltpu.VMEM((SEDGES, width), F32),
            pltpu.SemaphoreType.DMA, pltpu.SemaphoreType.DMA,
        ],
        compiler_params=_SC_PARAMS,
    )
    def edge_kernel(row_hbm, col_hbm, wz_hbm, gp_hbm, z_hbm, acc_out,
                    acc_sh, rowv0, rowv1, colv0, colv1, wzv0, wzv1,
                    rows0, rows1, sem_l, sem_g):
        c = lax.axis_index("c")
        s = lax.axis_index("s")
        wid = c * NS + s
        pltpu.sync_copy(z_hbm, acc_sh.at[pl.ds(s * stripe, stripe)])
        plsc.subcore_barrier()
        rowv = (rowv0, rowv1)
        colv = (colv0, colv1)
        wzv = (wzv0, wzv1)
        rows = (rows0, rows1)
        base0 = wid * (per_tile // CHUNK)

        def issue_loads(u, p):
            sl = pl.ds(base0 + u * SUPER, SUPER)
            pltpu.async_copy(row_hbm.at[sl], rowv[p], sem_l)
            pltpu.async_copy(col_hbm.at[sl], colv[p], sem_l)
            pltpu.async_copy(wz_hbm.at[sl], wzv[p], sem_l)

        def wait_loads(u, p):
            sl = pl.ds(base0 + u * SUPER, SUPER)
            pltpu.make_async_copy(row_hbm.at[sl], rowv[p], sem_l).wait()
            pltpu.make_async_copy(col_hbm.at[sl], colv[p], sem_l).wait()
            pltpu.make_async_copy(wz_hbm.at[sl], wzv[p], sem_l).wait()

        def fire_gathers(p):
            for k in range(SUPER):
                pltpu.async_copy(gp_hbm.at[rowv[p].at[k]],
                                 rows[p].at[pl.ds(k * CHUNK, CHUNK)], sem_g)

        def wait_gathers(p):
            for k in range(SUPER):
                pltpu.make_async_copy(
                    gp_hbm.at[rowv[p].at[k]],
                    rows[p].at[pl.ds(k * CHUNK, CHUNK)], sem_g).wait()

        def compute_scatter(p):
            rv = rows[p]
            for k in range(SUPER):
                for j in range(CHUNK // LANES):
                    wvec = wzv[p][k, pl.ds(j * LANES, LANES)]
                    for e in range(LANES):
                        ee = k * CHUNK + j * LANES + e
                        rv[ee] = rv[ee] * _splat(wvec, e)
                pltpu.sync_copy(rv.at[pl.ds(k * CHUNK, CHUNK)],
                                acc_sh.at[colv[p].at[k]], add=True)

        issue_loads(0, 0)
        wait_loads(0, 0)
        fire_gathers(0)
        issue_loads(1, 1)

        def step(u, p):
            @pl.when(u + 1 < n_steps)
            def _():
                wait_loads(u + 1, 1 - p)
                fire_gathers(1 - p)
            wait_gathers(p)
            compute_scatter(p)
            @pl.when(u + 2 < n_steps)
            def _():
                issue_loads(u + 2, p)

        def round_(r, carry):
            step(2 * r, 0)
            step(2 * r + 1, 1)
            return carry

        lax.fori_loop(0, n_steps // 2, round_, 0)
        plsc.subcore_barrier()
        pltpu.sync_copy(acc_sh.at[pl.ds(s * stripe, stripe)],
                        acc_out.at[pl.ds(c * n_pad + s * stripe, stripe)])

    return edge_kernel(row2, col2, wz2, gp,
                       jnp.zeros((stripe, width), F32))


_DOT = functools.partial(
    lax.dot_general,
    precision=lax.Precision.HIGHEST,
    preferred_element_type=F32,
)
_DN = (((1,), (1,)), ((), ()))


def _tc_mm_prep(x, Wa, Wb, deg_parts, blk=1024):
    n_pad, f = x.shape
    w = Wa.shape[0]
    nc, _, lanes = deg_parts.shape

    def body(x_ref, wa_ref, wb_ref, d_ref, oa_ref, dinv_ref, gp_ref):
        xb = x_ref[...]
        deg = jnp.sum(d_ref[...], axis=(0, 2), keepdims=False)[:, None]
        pos = deg > 0.0
        dinv = jnp.where(pos, lax.rsqrt(jnp.where(pos, deg, 1.0)), 0.0)
        dinv_bc = jnp.broadcast_to(dinv, (blk, w))
        oa_ref[...] = _DOT(xb, wa_ref[...], _DN)
        dinv_ref[...] = dinv_bc
        gp_ref[...] = dinv_bc * _DOT(xb, wb_ref[...], _DN)

    return pl.pallas_call(
        body,
        grid=(n_pad // blk,),
        in_specs=[
            pl.BlockSpec((blk, f), lambda i: (i, 0)),
            pl.BlockSpec((w, f), lambda i: (0, 0)),
            pl.BlockSpec((w, f), lambda i: (0, 0)),
            pl.BlockSpec((nc, blk, lanes), lambda i: (0, i, 0)),
        ],
        out_specs=[
            pl.BlockSpec((blk, w), lambda i: (i, 0)),
            pl.BlockSpec((blk, w), lambda i: (i, 0)),
            pl.BlockSpec((blk, w), lambda i: (i, 0)),
        ],
        out_shape=[
            jax.ShapeDtypeStruct((n_pad, w), F32),
            jax.ShapeDtypeStruct((n_pad, w), F32),
            jax.ShapeDtypeStruct((n_pad, w), F32),
        ],
    )(x, Wa, Wb, deg_parts)


def _tc_fuse_mid(g1a, s1_parts, dinv_bc, b1, W2a, W2b, blk=1024):
    n_pad, hid = g1a.shape
    w2 = W2a.shape[0]

    def body(ga_ref, s_ref, dinv_ref, b_ref, wa_ref, wb_ref, oa_ref, ogp_ref):
        dinv = dinv_ref[...]
        h = ga_ref[...] - dinv * jnp.sum(s_ref[...], axis=0) + b_ref[...]
        h = jnp.maximum(h, 0.0)
        oa_ref[...] = _DOT(h, wa_ref[...], _DN)
        ogp_ref[...] = dinv * _DOT(h, wb_ref[...], _DN)

    return pl.pallas_call(
        body,
        grid=(n_pad // blk,),
        in_specs=[
            pl.BlockSpec((blk, hid), lambda i: (i, 0)),
            pl.BlockSpec((NC, blk, hid), lambda i: (0, i, 0)),
            pl.BlockSpec((blk, hid), lambda i: (i, 0)),
            pl.BlockSpec((1, hid), lambda i: (0, 0)),
            pl.BlockSpec((w2, hid), lambda i: (0, 0)),
            pl.BlockSpec((w2, hid), lambda i: (0, 0)),
        ],
        out_specs=[
            pl.BlockSpec((blk, w2), lambda i: (i, 0)),
            pl.BlockSpec((blk, w2), lambda i: (i, 0)),
        ],
        out_shape=[
            jax.ShapeDtypeStruct((n_pad, w2), F32),
            jax.ShapeDtypeStruct((n_pad, w2), F32),
        ],
    )(g1a, s1_parts, dinv_bc, b1, W2a, W2b)


def _tc_fuse_out(g2a, s2_parts, dinv_bc, b2, blk=1024):
    n_pad, ncls = g2a.shape

    def body(ga_ref, s_ref, dinv_ref, b_ref, o_ref):
        z = (ga_ref[...] - dinv_ref[...] * jnp.sum(s_ref[...], axis=0)
             + b_ref[...])
        m = jnp.max(z, axis=1, keepdims=True)
        zm = z - m
        o_ref[...] = zm - jnp.log(jnp.sum(jnp.exp(zm), axis=1, keepdims=True))

    return pl.pallas_call(
        body,
        grid=(n_pad // blk,),
        in_specs=[
            pl.BlockSpec((blk, ncls), lambda i: (i, 0)),
            pl.BlockSpec((NC, blk, ncls), lambda i: (0, i, 0)),
            pl.BlockSpec((blk, ncls), lambda i: (i, 0)),
            pl.BlockSpec((1, ncls), lambda i: (0, 0)),
        ],
        out_specs=pl.BlockSpec((blk, ncls), lambda i: (i, 0)),
        out_shape=jax.ShapeDtypeStruct((n_pad, ncls), F32),
    )(g2a, s2_parts, dinv_bc, b2)


def kernel(x, edge_index, edge_attr, W1_0, W1_1, b1, W2_0, W2_1, b2):
    n, f_in = x.shape
    e = edge_attr.shape[0]
    hid = W1_0.shape[0]
    ncls = W2_0.shape[0]

    n_pad = _round_up(n, NS * 128)
    e_pad = _round_up(e, NW * SEDGES * 2)

    row_p = jnp.pad(edge_index[0], (0, e_pad - e))
    col_p = jnp.pad(edge_index[1], (0, e_pad - e))
    w_p = jnp.pad(edge_attr, (0, e_pad - e))
    x_pad = jnp.pad(x, ((0, n_pad - n), (0, 0)))

    row2 = row_p.reshape(-1, CHUNK)
    col2 = col_p.reshape(-1, CHUNK)
    deg_parts, wz2 = _sc_deg(row2, col2, w_p.reshape(-1, CHUNK), n_pad)
    g1a, dinv_bc, gp1 = _tc_mm_prep(x_pad, W1_0, W1_1,
                                    deg_parts.reshape(NC, n_pad, LANES))

    s1_flat = _sc_edge(row2, col2, wz2, gp1)
    g2a, gp2 = _tc_fuse_mid(g1a, s1_flat.reshape(NC, n_pad, hid), dinv_bc,
                            b1.reshape(1, hid), W2_0, W2_1)
    s2_flat = _sc_edge(row2, col2, wz2, gp2)
    out = _tc_fuse_out(g2a, s2_flat.reshape(NC, n_pad, ncls), dinv_bc,
                       b2.reshape(1, ncls))
    return out[:n]


# speedup vs baseline: 35.9858x; 1.2149x over previous
import functools

import jax
import jax.numpy as jnp
from jax import lax
from jax.experimental import pallas as pl
from jax.experimental.pallas import tpu as pltpu
from jax.experimental.pallas import tpu_sc as plsc

NC = 2
NS = 16
NW = NC * NS
LANES = 16
CHUNK = 128

F32 = jnp.float32
I32 = jnp.int32

_SC_PARAMS = pltpu.CompilerParams(use_tc_tiling_on_sc=False)


def _round_up(v, m):
    return (v + m - 1) // m * m


def _mesh():
    return plsc.VectorSubcoreMesh(core_axis_name="c", subcore_axis_name="s",
                                  num_cores=NC, num_subcores=NS)


def _splat(vec, e):
    idx = jnp.full((LANES,), e, I32)
    return lax.gather(
        vec, idx[:, None],
        lax.GatherDimensionNumbers(offset_dims=(), collapsed_slice_dims=(0,),
                                   start_index_map=(0,)),
        (1,), mode=lax.GatherScatterMode.PROMISE_IN_BOUNDS)


def _sc_deg(row2, col2, w2, n_pad):
    t_rows = row2.shape[0]
    e_pad = t_rows * CHUNK
    per_tile = e_pad // NW
    n_steps = per_tile // SEDGES
    assert n_steps % 2 == 0 and n_steps >= 4
    stripe = n_pad // NS

    @functools.partial(
        pl.kernel,
        out_type=(jax.ShapeDtypeStruct((NC * n_pad, LANES), F32),
                  jax.ShapeDtypeStruct((t_rows, CHUNK), F32)),
        mesh=_mesh(),
        scratch_types=[
            pltpu.VMEM_SHARED((n_pad, LANES), F32),
            pltpu.VMEM((SUPER, CHUNK), I32), pltpu.VMEM((SUPER, CHUNK), I32),
            pltpu.VMEM((SUPER, CHUNK), I32), pltpu.VMEM((SUPER, CHUNK), I32),
            pltpu.VMEM((SUPER, CHUNK), F32), pltpu.VMEM((SUPER, CHUNK), F32),
            pltpu.VMEM((SUPER, CHUNK), F32), pltpu.VMEM((SUPER, CHUNK), F32),
            pltpu.VMEM((SEDGES, LANES), F32),
            pltpu.SemaphoreType.DMA, pltpu.SemaphoreType.DMA,
        ],
        compiler_params=_SC_PARAMS,
    )
    def deg_kernel(row_hbm, col_hbm, w_hbm, z_hbm, deg_out, wz_out,
                   acc_sh, rowv0, rowv1, colv0, colv1, wv0, wv1,
                   wzv0, wzv1, valv, sem_l, sem_w):
        c = lax.axis_index("c")
        s = lax.axis_index("s")
        wid = c * NS + s
        pltpu.sync_copy(z_hbm, acc_sh.at[pl.ds(s * stripe, stripe)])
        plsc.subcore_barrier()
        rowv = (rowv0, rowv1)
        colv = (colv0, colv1)
        wv = (wv0, wv1)
        wzv = (wzv0, wzv1)
        base0 = wid * (per_tile // CHUNK)
        iota = lax.broadcasted_iota(I32, (LANES,), 0)

        def issue_loads(u, p):
            sl = pl.ds(base0 + u * SUPER, SUPER)
            pltpu.async_copy(row_hbm.at[sl], rowv[p], sem_l)
            pltpu.async_copy(col_hbm.at[sl], colv[p], sem_l)
            pltpu.async_copy(w_hbm.at[sl], wv[p], sem_l)

        def wait_loads(u, p):
            sl = pl.ds(base0 + u * SUPER, SUPER)
            pltpu.make_async_copy(row_hbm.at[sl], rowv[p], sem_l).wait()
            pltpu.make_async_copy(col_hbm.at[sl], colv[p], sem_l).wait()
            pltpu.make_async_copy(w_hbm.at[sl], wv[p], sem_l).wait()

        def wz_slice(u):
            return pl.ds(base0 + u * SUPER, SUPER)

        issue_loads(0, 0)
        wait_loads(0, 0)
        issue_loads(1, 1)

        def step(u, p):
            @pl.when(u >= 2)
            def _():
                pltpu.make_async_copy(wzv[p], wz_out.at[wz_slice(u - 2)],
                                      sem_w).wait()
            for k in range(SUPER):
                for j in range(CHUNK // LANES):
                    sl = pl.ds(j * LANES, LANES)
                    wz = jnp.where(rowv[p][k, sl] == colv[p][k, sl],
                                   0.0, wv[p][k, sl])
                    wzv[p][k, sl] = wz
                    for e in range(LANES):
                        valv[k * CHUNK + j * LANES + e] = (
                            jnp.where(iota == e, wz, 0.0))
                pltpu.sync_copy(valv.at[pl.ds(k * CHUNK, CHUNK)],
                                acc_sh.at[rowv[p].at[k]], add=True)
            pltpu.async_copy(wzv[p], wz_out.at[wz_slice(u)], sem_w)
            @pl.when(u + 1 < n_steps)
            def _():
                wait_loads(u + 1, 1 - p)
            @pl.when(u + 2 < n_steps)
            def _():
                issue_loads(u + 2, p)

        def round_(r, carry):
            step(2 * r, 0)
            step(2 * r + 1, 1)
            return carry

        lax.fori_loop(0, n_steps // 2, round_, 0)
        pltpu.make_async_copy(wzv[0], wz_out.at[wz_slice(n_steps - 2)],
                              sem_w).wait()
        pltpu.make_async_copy(wzv[1], wz_out.at[wz_slice(n_steps - 1)],
                              sem_w).wait()
        plsc.subcore_barrier()
        pltpu.sync_copy(acc_sh.at[pl.ds(s * stripe, stripe)],
                        deg_out.at[pl.ds(c * n_pad + s * stripe, stripe)])

    return deg_kernel(row2, col2, w2, jnp.zeros((stripe, LANES), F32))


SUPER = 4
SEDGES = SUPER * CHUNK


def _sc_edge(row2, col2, wz2, gp):
    t_rows = row2.shape[0]
    e_pad = t_rows * CHUNK
    n_pad, width = gp.shape
    per_tile = e_pad // NW
    n_steps = per_tile // SEDGES
    assert n_steps % 2 == 0 and n_steps >= 4
    stripe = n_pad // NS

    @functools.partial(
        pl.kernel,
        out_type=jax.ShapeDtypeStruct((NC * n_pad, width), F32),
        mesh=_mesh(),
        scratch_types=[
            pltpu.VMEM_SHARED((n_pad, width), F32),
            pltpu.VMEM_SHARED((n_pad, width), F32),
            pltpu.VMEM((SUPER, CHUNK), I32), pltpu.VMEM((SUPER, CHUNK), I32),
            pltpu.VMEM((SUPER, CHUNK), I32), pltpu.VMEM((SUPER, CHUNK), I32),
            pltpu.VMEM((SUPER, CHUNK), F32), pltpu.VMEM((SUPER, CHUNK), F32),
            pltpu.VMEM((SEDGES, width), F32), pltpu.VMEM((SEDGES, width), F32),
            pltpu.SemaphoreType.DMA, pltpu.SemaphoreType.DMA,
        ],
        compiler_params=_SC_PARAMS,
    )
    def edge_kernel(row_hbm, col_hbm, wz_hbm, gp_hbm, z_hbm, acc_out,
                    acc_sh, gp_sh, rowv0, rowv1, colv0, colv1, wzv0, wzv1,
                    rows0, rows1, sem_l, sem_g):
        c = lax.axis_index("c")
        s = lax.axis_index("s")
        wid = c * NS + s
        pltpu.sync_copy(z_hbm, acc_sh.at[pl.ds(s * stripe, stripe)])
        pltpu.sync_copy(gp_hbm.at[pl.ds(s * stripe, stripe)],
                        gp_sh.at[pl.ds(s * stripe, stripe)])
        plsc.subcore_barrier()
        rowv = (rowv0, rowv1)
        colv = (colv0, colv1)
        wzv = (wzv0, wzv1)
        rows = (rows0, rows1)
        base0 = wid * (per_tile // CHUNK)

        def issue_loads(u, p):
            sl = pl.ds(base0 + u * SUPER, SUPER)
            pltpu.async_copy(row_hbm.at[sl], rowv[p], sem_l)
            pltpu.async_copy(col_hbm.at[sl], colv[p], sem_l)
            pltpu.async_copy(wz_hbm.at[sl], wzv[p], sem_l)

        def wait_loads(u, p):
            sl = pl.ds(base0 + u * SUPER, SUPER)
            pltpu.make_async_copy(row_hbm.at[sl], rowv[p], sem_l).wait()
            pltpu.make_async_copy(col_hbm.at[sl], colv[p], sem_l).wait()
            pltpu.make_async_copy(wz_hbm.at[sl], wzv[p], sem_l).wait()

        def fire_gathers(p):
            for k in range(SUPER):
                pltpu.async_copy(gp_sh.at[rowv[p].at[k]],
                                 rows[p].at[pl.ds(k * CHUNK, CHUNK)], sem_g)

        def wait_gathers(p):
            for k in range(SUPER):
                pltpu.make_async_copy(
                    gp_sh.at[rowv[p].at[k]],
                    rows[p].at[pl.ds(k * CHUNK, CHUNK)], sem_g).wait()

        def compute_scatter(p):
            rv = rows[p]
            for k in range(SUPER):
                for j in range(CHUNK // LANES):
                    wvec = wzv[p][k, pl.ds(j * LANES, LANES)]
                    for e in range(LANES):
                        ee = k * CHUNK + j * LANES + e
                        rv[ee] = rv[ee] * _splat(wvec, e)
                pltpu.sync_copy(rv.at[pl.ds(k * CHUNK, CHUNK)],
                                acc_sh.at[colv[p].at[k]], add=True)

        issue_loads(0, 0)
        wait_loads(0, 0)
        fire_gathers(0)
        issue_loads(1, 1)

        def step(u, p):
            @pl.when(u + 1 < n_steps)
            def _():
                wait_loads(u + 1, 1 - p)
                fire_gathers(1 - p)
            wait_gathers(p)
            compute_scatter(p)
            @pl.when(u + 2 < n_steps)
            def _():
                issue_loads(u + 2, p)

        def round_(r, carry):
            step(2 * r, 0)
            step(2 * r + 1, 1)
            return carry

        lax.fori_loop(0, n_steps // 2, round_, 0)
        plsc.subcore_barrier()
        pltpu.sync_copy(acc_sh.at[pl.ds(s * stripe, stripe)],
                        acc_out.at[pl.ds(c * n_pad + s * stripe, stripe)])

    return edge_kernel(row2, col2, wz2, gp,
                       jnp.zeros((stripe, width), F32))


_DOT = functools.partial(
    lax.dot_general,
    precision=lax.Precision.HIGHEST,
    preferred_element_type=F32,
)
_DN = (((1,), (1,)), ((), ()))


def _tc_mm_prep(x, Wa, Wb, deg_parts, blk=1024):
    n_pad, f = x.shape
    w = Wa.shape[0]
    nc, _, lanes = deg_parts.shape

    def body(x_ref, wa_ref, wb_ref, d_ref, oa_ref, dinv_ref, gp_ref):
        xb = x_ref[...]
        deg = jnp.sum(d_ref[...], axis=(0, 2), keepdims=False)[:, None]
        pos = deg > 0.0
        dinv = jnp.where(pos, lax.rsqrt(jnp.where(pos, deg, 1.0)), 0.0)
        dinv_bc = jnp.broadcast_to(dinv, (blk, w))
        oa_ref[...] = _DOT(xb, wa_ref[...], _DN)
        dinv_ref[...] = dinv_bc
        gp_ref[...] = dinv_bc * _DOT(xb, wb_ref[...], _DN)

    return pl.pallas_call(
        body,
        grid=(n_pad // blk,),
        in_specs=[
            pl.BlockSpec((blk, f), lambda i: (i, 0)),
            pl.BlockSpec((w, f), lambda i: (0, 0)),
            pl.BlockSpec((w, f), lambda i: (0, 0)),
            pl.BlockSpec((nc, blk, lanes), lambda i: (0, i, 0)),
        ],
        out_specs=[
            pl.BlockSpec((blk, w), lambda i: (i, 0)),
            pl.BlockSpec((blk, w), lambda i: (i, 0)),
            pl.BlockSpec((blk, w), lambda i: (i, 0)),
        ],
        out_shape=[
            jax.ShapeDtypeStruct((n_pad, w), F32),
            jax.ShapeDtypeStruct((n_pad, w), F32),
            jax.ShapeDtypeStruct((n_pad, w), F32),
        ],
    )(x, Wa, Wb, deg_parts)


def _tc_fuse_mid(g1a, s1_parts, dinv_bc, b1, W2a, W2b, blk=1024):
    n_pad, hid = g1a.shape
    w2 = W2a.shape[0]

    def body(ga_ref, s_ref, dinv_ref, b_ref, wa_ref, wb_ref, oa_ref, ogp_ref):
        dinv = dinv_ref[...]
        h = ga_ref[...] - dinv * jnp.sum(s_ref[...], axis=0) + b_ref[...]
        h = jnp.maximum(h, 0.0)
        oa_ref[...] = _DOT(h, wa_ref[...], _DN)
        ogp_ref[...] = dinv * _DOT(h, wb_ref[...], _DN)

    return pl.pallas_call(
        body,
        grid=(n_pad // blk,),
        in_specs=[
            pl.BlockSpec((blk, hid), lambda i: (i, 0)),
            pl.BlockSpec((NC, blk, hid), lambda i: (0, i, 0)),
            pl.BlockSpec((blk, hid), lambda i: (i, 0)),
            pl.BlockSpec((1, hid), lambda i: (0, 0)),
            pl.BlockSpec((w2, hid), lambda i: (0, 0)),
            pl.BlockSpec((w2, hid), lambda i: (0, 0)),
        ],
        out_specs=[
            pl.BlockSpec((blk, w2), lambda i: (i, 0)),
            pl.BlockSpec((blk, w2), lambda i: (i, 0)),
        ],
        out_shape=[
            jax.ShapeDtypeStruct((n_pad, w2), F32),
            jax.ShapeDtypeStruct((n_pad, w2), F32),
        ],
    )(g1a, s1_parts, dinv_bc, b1, W2a, W2b)


def _tc_fuse_out(g2a, s2_parts, dinv_bc, b2, blk=1024):
    n_pad, ncls = g2a.shape

    def body(ga_ref, s_ref, dinv_ref, b_ref, o_ref):
        z = (ga_ref[...] - dinv_ref[...] * jnp.sum(s_ref[...], axis=0)
             + b_ref[...])
        m = jnp.max(z, axis=1, keepdims=True)
        zm = z - m
        o_ref[...] = zm - jnp.log(jnp.sum(jnp.exp(zm), axis=1, keepdims=True))

    return pl.pallas_call(
        body,
        grid=(n_pad // blk,),
        in_specs=[
            pl.BlockSpec((blk, ncls), lambda i: (i, 0)),
            pl.BlockSpec((NC, blk, ncls), lambda i: (0, i, 0)),
            pl.BlockSpec((blk, ncls), lambda i: (i, 0)),
            pl.BlockSpec((1, ncls), lambda i: (0, 0)),
        ],
        out_specs=pl.BlockSpec((blk, ncls), lambda i: (i, 0)),
        out_shape=jax.ShapeDtypeStruct((n_pad, ncls), F32),
    )(g2a, s2_parts, dinv_bc, b2)


def kernel(x, edge_index, edge_attr, W1_0, W1_1, b1, W2_0, W2_1, b2):
    n, f_in = x.shape
    e = edge_attr.shape[0]
    hid = W1_0.shape[0]
    ncls = W2_0.shape[0]

    n_pad = _round_up(n, NS * 128)
    e_pad = _round_up(e, NW * SEDGES * 2)

    row_p = jnp.pad(edge_index[0], (0, e_pad - e))
    col_p = jnp.pad(edge_index[1], (0, e_pad - e))
    w_p = jnp.pad(edge_attr, (0, e_pad - e))
    x_pad = jnp.pad(x, ((0, n_pad - n), (0, 0)))

    row2 = row_p.reshape(-1, CHUNK)
    col2 = col_p.reshape(-1, CHUNK)
    deg_parts, wz2 = _sc_deg(row2, col2, w_p.reshape(-1, CHUNK), n_pad)
    g1a, dinv_bc, gp1 = _tc_mm_prep(x_pad, W1_0, W1_1,
                                    deg_parts.reshape(NC, n_pad, LANES))

    s1_flat = _sc_edge(row2, col2, wz2, gp1)
    g2a, gp2 = _tc_fuse_mid(g1a, s1_flat.reshape(NC, n_pad, hid), dinv_bc,
                            b1.reshape(1, hid), W2_0, W2_1)
    s2_flat = _sc_edge(row2, col2, wz2, gp2)
    out = _tc_fuse_out(g2a, s2_flat.reshape(NC, n_pad, ncls), dinv_bc,
                       b2.reshape(1, ncls))
    return out[:n]


# speedup vs baseline: 36.5626x; 1.0160x over previous
import functools

import jax
import jax.numpy as jnp
from jax import lax
from jax.experimental import pallas as pl
from jax.experimental.pallas import tpu as pltpu
from jax.experimental.pallas import tpu_sc as plsc

NC = 2
NS = 16
NW = NC * NS
LANES = 16
CHUNK = 128

F32 = jnp.float32
I32 = jnp.int32

_SC_PARAMS = pltpu.CompilerParams(use_tc_tiling_on_sc=False)


def _round_up(v, m):
    return (v + m - 1) // m * m


def _mesh():
    return plsc.VectorSubcoreMesh(core_axis_name="c", subcore_axis_name="s",
                                  num_cores=NC, num_subcores=NS)


def _splat(vec, e):
    idx = jnp.full((LANES,), e, I32)
    return lax.gather(
        vec, idx[:, None],
        lax.GatherDimensionNumbers(offset_dims=(), collapsed_slice_dims=(0,),
                                   start_index_map=(0,)),
        (1,), mode=lax.GatherScatterMode.PROMISE_IN_BOUNDS)


def _sc_deg(row2, col2, w2, n_pad):
    t_rows = row2.shape[0]
    e_pad = t_rows * CHUNK
    per_tile = e_pad // NW
    n_steps = per_tile // SEDGES
    assert n_steps % 2 == 0 and n_steps >= 4
    stripe = n_pad // NS

    @functools.partial(
        pl.kernel,
        out_type=(jax.ShapeDtypeStruct((NC * n_pad, LANES), F32),
                  jax.ShapeDtypeStruct((t_rows, CHUNK), F32)),
        mesh=_mesh(),
        scratch_types=[
            pltpu.VMEM_SHARED((n_pad, LANES), F32),
            pltpu.VMEM((SUPER, CHUNK), I32), pltpu.VMEM((SUPER, CHUNK), I32),
            pltpu.VMEM((SUPER, CHUNK), I32), pltpu.VMEM((SUPER, CHUNK), I32),
            pltpu.VMEM((SUPER, CHUNK), F32), pltpu.VMEM((SUPER, CHUNK), F32),
            pltpu.VMEM((SUPER, CHUNK), F32), pltpu.VMEM((SUPER, CHUNK), F32),
            pltpu.VMEM((SEDGES, LANES), F32),
            pltpu.SemaphoreType.DMA, pltpu.SemaphoreType.DMA,
        ],
        compiler_params=_SC_PARAMS,
    )
    def deg_kernel(row_hbm, col_hbm, w_hbm, z_hbm, deg_out, wz_out,
                   acc_sh, rowv0, rowv1, colv0, colv1, wv0, wv1,
                   wzv0, wzv1, valv, sem_l, sem_w):
        c = lax.axis_index("c")
        s = lax.axis_index("s")
        wid = c * NS + s
        pltpu.sync_copy(z_hbm, acc_sh.at[pl.ds(s * stripe, stripe)])
        plsc.subcore_barrier()
        rowv = (rowv0, rowv1)
        colv = (colv0, colv1)
        wv = (wv0, wv1)
        wzv = (wzv0, wzv1)
        base0 = wid * (per_tile // CHUNK)
        iota = lax.broadcasted_iota(I32, (LANES,), 0)

        def issue_loads(u, p):
            sl = pl.ds(base0 + u * SUPER, SUPER)
            pltpu.async_copy(row_hbm.at[sl], rowv[p], sem_l)
            pltpu.async_copy(col_hbm.at[sl], colv[p], sem_l)
            pltpu.async_copy(w_hbm.at[sl], wv[p], sem_l)

        def wait_loads(u, p):
            sl = pl.ds(base0 + u * SUPER, SUPER)
            pltpu.make_async_copy(row_hbm.at[sl], rowv[p], sem_l).wait()
            pltpu.make_async_copy(col_hbm.at[sl], colv[p], sem_l).wait()
            pltpu.make_async_copy(w_hbm.at[sl], wv[p], sem_l).wait()

        def wz_slice(u):
            return pl.ds(base0 + u * SUPER, SUPER)

        issue_loads(0, 0)
        wait_loads(0, 0)
        issue_loads(1, 1)

        def step(u, p):
            @pl.when(u >= 2)
            def _():
                pltpu.make_async_copy(wzv[p], wz_out.at[wz_slice(u - 2)],
                                      sem_w).wait()
            for k in range(SUPER):
                for j in range(CHUNK // LANES):
                    sl = pl.ds(j * LANES, LANES)
                    wz = jnp.where(rowv[p][k, sl] == colv[p][k, sl],
                                   0.0, wv[p][k, sl])
                    wzv[p][k, sl] = wz
                    for e in range(LANES):
                        valv[k * CHUNK + j * LANES + e] = (
                            jnp.where(iota == e, wz, 0.0))
                pltpu.sync_copy(valv.at[pl.ds(k * CHUNK, CHUNK)],
                                acc_sh.at[rowv[p].at[k]], add=True)
            pltpu.async_copy(wzv[p], wz_out.at[wz_slice(u)], sem_w)
            @pl.when(u + 1 < n_steps)
            def _():
                wait_loads(u + 1, 1 - p)
            @pl.when(u + 2 < n_steps)
            def _():
                issue_loads(u + 2, p)

        def round_(r, carry):
            step(2 * r, 0)
            step(2 * r + 1, 1)
            return carry

        lax.fori_loop(0, n_steps // 2, round_, 0)
        pltpu.make_async_copy(wzv[0], wz_out.at[wz_slice(n_steps - 2)],
                              sem_w).wait()
        pltpu.make_async_copy(wzv[1], wz_out.at[wz_slice(n_steps - 1)],
                              sem_w).wait()
        plsc.subcore_barrier()
        pltpu.sync_copy(acc_sh.at[pl.ds(s * stripe, stripe)],
                        deg_out.at[pl.ds(c * n_pad + s * stripe, stripe)])

    return deg_kernel(row2, col2, w2, jnp.zeros((stripe, LANES), F32))


SUPER = 4
SEDGES = SUPER * CHUNK
SUPER_E = 8
SEDGES_E = SUPER_E * CHUNK


def _sc_edge(row2, col2, wz2, gp):
    t_rows = row2.shape[0]
    e_pad = t_rows * CHUNK
    n_pad, width = gp.shape
    per_tile = e_pad // NW
    n_steps = per_tile // SEDGES_E
    assert n_steps % 2 == 0 and n_steps >= 4
    stripe = n_pad // NS

    @functools.partial(
        pl.kernel,
        out_type=jax.ShapeDtypeStruct((NC * n_pad, width), F32),
        mesh=_mesh(),
        scratch_types=[
            pltpu.VMEM_SHARED((n_pad, width), F32),
            pltpu.VMEM_SHARED((n_pad, width), F32),
            pltpu.VMEM((SUPER_E, CHUNK), I32), pltpu.VMEM((SUPER_E, CHUNK), I32),
            pltpu.VMEM((SUPER_E, CHUNK), I32), pltpu.VMEM((SUPER_E, CHUNK), I32),
            pltpu.VMEM((SUPER_E, CHUNK), F32), pltpu.VMEM((SUPER_E, CHUNK), F32),
            pltpu.VMEM((SEDGES_E, width), F32), pltpu.VMEM((SEDGES_E, width), F32),
            pltpu.SemaphoreType.DMA, pltpu.SemaphoreType.DMA,
        ],
        compiler_params=_SC_PARAMS,
    )
    def edge_kernel(row_hbm, col_hbm, wz_hbm, gp_hbm, z_hbm, acc_out,
                    acc_sh, gp_sh, rowv0, rowv1, colv0, colv1, wzv0, wzv1,
                    rows0, rows1, sem_l, sem_g):
        c = lax.axis_index("c")
        s = lax.axis_index("s")
        wid = c * NS + s
        pltpu.sync_copy(z_hbm, acc_sh.at[pl.ds(s * stripe, stripe)])
        pltpu.sync_copy(gp_hbm.at[pl.ds(s * stripe, stripe)],
                        gp_sh.at[pl.ds(s * stripe, stripe)])
        plsc.subcore_barrier()
        rowv = (rowv0, rowv1)
        colv = (colv0, colv1)
        wzv = (wzv0, wzv1)
        rows = (rows0, rows1)
        base0 = wid * (per_tile // CHUNK)

        def issue_loads(u, p):
            sl = pl.ds(base0 + u * SUPER_E, SUPER_E)
            pltpu.async_copy(row_hbm.at[sl], rowv[p], sem_l)
            pltpu.async_copy(col_hbm.at[sl], colv[p], sem_l)
            pltpu.async_copy(wz_hbm.at[sl], wzv[p], sem_l)

        def wait_loads(u, p):
            sl = pl.ds(base0 + u * SUPER_E, SUPER_E)
            pltpu.make_async_copy(row_hbm.at[sl], rowv[p], sem_l).wait()
            pltpu.make_async_copy(col_hbm.at[sl], colv[p], sem_l).wait()
            pltpu.make_async_copy(wz_hbm.at[sl], wzv[p], sem_l).wait()

        def fire_gathers(p):
            for k in range(SUPER_E):
                pltpu.async_copy(gp_sh.at[rowv[p].at[k]],
                                 rows[p].at[pl.ds(k * CHUNK, CHUNK)], sem_g)

        def wait_gathers(p):
            for k in range(SUPER_E):
                pltpu.make_async_copy(
                    gp_sh.at[rowv[p].at[k]],
                    rows[p].at[pl.ds(k * CHUNK, CHUNK)], sem_g).wait()

        def compute_scatter(p):
            rv = rows[p]
            for k in range(SUPER_E):
                for j in range(CHUNK // LANES):
                    wvec = wzv[p][k, pl.ds(j * LANES, LANES)]
                    for e in range(LANES):
                        ee = k * CHUNK + j * LANES + e
                        rv[ee] = rv[ee] * _splat(wvec, e)
                pltpu.sync_copy(rv.at[pl.ds(k * CHUNK, CHUNK)],
                                acc_sh.at[colv[p].at[k]], add=True)

        issue_loads(0, 0)
        wait_loads(0, 0)
        fire_gathers(0)
        issue_loads(1, 1)

        def step(u, p):
            @pl.when(u + 1 < n_steps)
            def _():
                wait_loads(u + 1, 1 - p)
                fire_gathers(1 - p)
            wait_gathers(p)
            compute_scatter(p)
            @pl.when(u + 2 < n_steps)
            def _():
                issue_loads(u + 2, p)

        def round_(r, carry):
            step(2 * r, 0)
            step(2 * r + 1, 1)
            return carry

        lax.fori_loop(0, n_steps // 2, round_, 0)
        plsc.subcore_barrier()
        pltpu.sync_copy(acc_sh.at[pl.ds(s * stripe, stripe)],
                        acc_out.at[pl.ds(c * n_pad + s * stripe, stripe)])

    return edge_kernel(row2, col2, wz2, gp,
                       jnp.zeros((stripe, width), F32))


_DOT = functools.partial(
    lax.dot_general,
    precision=lax.Precision.HIGHEST,
    preferred_element_type=F32,
)
_DN = (((1,), (1,)), ((), ()))


def _tc_mm_prep(x, Wa, Wb, deg_parts, blk=1024):
    n_pad, f = x.shape
    w = Wa.shape[0]
    nc, _, lanes = deg_parts.shape

    def body(x_ref, wa_ref, wb_ref, d_ref, oa_ref, dinv_ref, gp_ref):
        xb = x_ref[...]
        deg = jnp.sum(d_ref[...], axis=(0, 2), keepdims=False)[:, None]
        pos = deg > 0.0
        dinv = jnp.where(pos, lax.rsqrt(jnp.where(pos, deg, 1.0)), 0.0)
        dinv_bc = jnp.broadcast_to(dinv, (blk, w))
        oa_ref[...] = _DOT(xb, wa_ref[...], _DN)
        dinv_ref[...] = dinv_bc
        gp_ref[...] = dinv_bc * _DOT(xb, wb_ref[...], _DN)

    return pl.pallas_call(
        body,
        grid=(n_pad // blk,),
        in_specs=[
            pl.BlockSpec((blk, f), lambda i: (i, 0)),
            pl.BlockSpec((w, f), lambda i: (0, 0)),
            pl.BlockSpec((w, f), lambda i: (0, 0)),
            pl.BlockSpec((nc, blk, lanes), lambda i: (0, i, 0)),
        ],
        out_specs=[
            pl.BlockSpec((blk, w), lambda i: (i, 0)),
            pl.BlockSpec((blk, w), lambda i: (i, 0)),
            pl.BlockSpec((blk, w), lambda i: (i, 0)),
        ],
        out_shape=[
            jax.ShapeDtypeStruct((n_pad, w), F32),
            jax.ShapeDtypeStruct((n_pad, w), F32),
            jax.ShapeDtypeStruct((n_pad, w), F32),
        ],
    )(x, Wa, Wb, deg_parts)


def _tc_fuse_mid(g1a, s1_parts, dinv_bc, b1, W2a, W2b, blk=1024):
    n_pad, hid = g1a.shape
    w2 = W2a.shape[0]

    def body(ga_ref, s_ref, dinv_ref, b_ref, wa_ref, wb_ref, oa_ref, ogp_ref):
        dinv = dinv_ref[...]
        h = ga_ref[...] - dinv * jnp.sum(s_ref[...], axis=0) + b_ref[...]
        h = jnp.maximum(h, 0.0)
        oa_ref[...] = _DOT(h, wa_ref[...], _DN)
        ogp_ref[...] = dinv * _DOT(h, wb_ref[...], _DN)

    return pl.pallas_call(
        body,
        grid=(n_pad // blk,),
        in_specs=[
            pl.BlockSpec((blk, hid), lambda i: (i, 0)),
            pl.BlockSpec((NC, blk, hid), lambda i: (0, i, 0)),
            pl.BlockSpec((blk, hid), lambda i: (i, 0)),
            pl.BlockSpec((1, hid), lambda i: (0, 0)),
            pl.BlockSpec((w2, hid), lambda i: (0, 0)),
            pl.BlockSpec((w2, hid), lambda i: (0, 0)),
        ],
        out_specs=[
            pl.BlockSpec((blk, w2), lambda i: (i, 0)),
            pl.BlockSpec((blk, w2), lambda i: (i, 0)),
        ],
        out_shape=[
            jax.ShapeDtypeStruct((n_pad, w2), F32),
            jax.ShapeDtypeStruct((n_pad, w2), F32),
        ],
    )(g1a, s1_parts, dinv_bc, b1, W2a, W2b)


def _tc_fuse_out(g2a, s2_parts, dinv_bc, b2, blk=1024):
    n_pad, ncls = g2a.shape

    def body(ga_ref, s_ref, dinv_ref, b_ref, o_ref):
        z = (ga_ref[...] - dinv_ref[...] * jnp.sum(s_ref[...], axis=0)
             + b_ref[...])
        m = jnp.max(z, axis=1, keepdims=True)
        zm = z - m
        o_ref[...] = zm - jnp.log(jnp.sum(jnp.exp(zm), axis=1, keepdims=True))

    return pl.pallas_call(
        body,
        grid=(n_pad // blk,),
        in_specs=[
            pl.BlockSpec((blk, ncls), lambda i: (i, 0)),
            pl.BlockSpec((NC, blk, ncls), lambda i: (0, i, 0)),
            pl.BlockSpec((blk, ncls), lambda i: (i, 0)),
            pl.BlockSpec((1, ncls), lambda i: (0, 0)),
        ],
        out_specs=pl.BlockSpec((blk, ncls), lambda i: (i, 0)),
        out_shape=jax.ShapeDtypeStruct((n_pad, ncls), F32),
    )(g2a, s2_parts, dinv_bc, b2)


def kernel(x, edge_index, edge_attr, W1_0, W1_1, b1, W2_0, W2_1, b2):
    n, f_in = x.shape
    e = edge_attr.shape[0]
    hid = W1_0.shape[0]
    ncls = W2_0.shape[0]

    n_pad = _round_up(n, NS * 128)
    e_pad = _round_up(e, NW * CHUNK * 16)

    row_p = jnp.pad(edge_index[0], (0, e_pad - e))
    col_p = jnp.pad(edge_index[1], (0, e_pad - e))
    w_p = jnp.pad(edge_attr, (0, e_pad - e))
    x_pad = jnp.pad(x, ((0, n_pad - n), (0, 0)))

    row2 = row_p.reshape(-1, CHUNK)
    col2 = col_p.reshape(-1, CHUNK)
    deg_parts, wz2 = _sc_deg(row2, col2, w_p.reshape(-1, CHUNK), n_pad)
    g1a, dinv_bc, gp1 = _tc_mm_prep(x_pad, W1_0, W1_1,
                                    deg_parts.reshape(NC, n_pad, LANES))

    s1_flat = _sc_edge(row2, col2, wz2, gp1)
    g2a, gp2 = _tc_fuse_mid(g1a, s1_flat.reshape(NC, n_pad, hid), dinv_bc,
                            b1.reshape(1, hid), W2_0, W2_1)
    s2_flat = _sc_edge(row2, col2, wz2, gp2)
    out = _tc_fuse_out(g2a, s2_flat.reshape(NC, n_pad, ncls), dinv_bc,
                       b2.reshape(1, ncls))
    return out[:n]


# speedup vs baseline: 37.4205x; 1.0235x over previous
import functools

import jax
import jax.numpy as jnp
from jax import lax
from jax.experimental import pallas as pl
from jax.experimental.pallas import tpu as pltpu
from jax.experimental.pallas import tpu_sc as plsc

NC = 2
NS = 16
NW = NC * NS
LANES = 16
CHUNK = 128

F32 = jnp.float32
I32 = jnp.int32

_SC_PARAMS = pltpu.CompilerParams(use_tc_tiling_on_sc=False)


def _round_up(v, m):
    return (v + m - 1) // m * m


def _mesh():
    return plsc.VectorSubcoreMesh(core_axis_name="c", subcore_axis_name="s",
                                  num_cores=NC, num_subcores=NS)


def _splat(vec, e):
    idx = jnp.full((LANES,), e, I32)
    return lax.gather(
        vec, idx[:, None],
        lax.GatherDimensionNumbers(offset_dims=(), collapsed_slice_dims=(0,),
                                   start_index_map=(0,)),
        (1,), mode=lax.GatherScatterMode.PROMISE_IN_BOUNDS)


def _sc_deg(row2, col2, w2, n_pad):
    t_rows = row2.shape[0]
    e_pad = t_rows * CHUNK
    per_tile = e_pad // NW
    n_steps = per_tile // SEDGES
    assert n_steps % 2 == 0 and n_steps >= 4
    stripe = n_pad // NS

    @functools.partial(
        pl.kernel,
        out_type=jax.ShapeDtypeStruct((NC * n_pad, LANES), F32),
        mesh=_mesh(),
        scratch_types=[
            pltpu.VMEM_SHARED((n_pad, LANES), F32),
            pltpu.VMEM((SUPER, CHUNK), I32), pltpu.VMEM((SUPER, CHUNK), I32),
            pltpu.VMEM((SUPER, CHUNK), I32), pltpu.VMEM((SUPER, CHUNK), I32),
            pltpu.VMEM((SUPER, CHUNK), F32), pltpu.VMEM((SUPER, CHUNK), F32),
            pltpu.VMEM((SEDGES, LANES), F32), pltpu.VMEM((SEDGES, LANES), F32),
            pltpu.SemaphoreType.DMA, pltpu.SemaphoreType.DMA,
        ],
        compiler_params=_SC_PARAMS,
    )
    def deg_kernel(row_hbm, col_hbm, w_hbm, z_hbm, deg_out,
                   acc_sh, rowv0, rowv1, colv0, colv1, wv0, wv1,
                   valv0, valv1, sem_l, sem_s):
        c = lax.axis_index("c")
        s = lax.axis_index("s")
        wid = c * NS + s
        pltpu.sync_copy(z_hbm, acc_sh.at[pl.ds(s * stripe, stripe)])
        plsc.subcore_barrier()
        rowv = (rowv0, rowv1)
        colv = (colv0, colv1)
        wv = (wv0, wv1)
        valv = (valv0, valv1)
        base0 = wid * (per_tile // CHUNK)
        iota = lax.broadcasted_iota(I32, (LANES,), 0)

        def issue_loads(u, p):
            sl = pl.ds(base0 + u * SUPER, SUPER)
            pltpu.async_copy(row_hbm.at[sl], rowv[p], sem_l)
            pltpu.async_copy(col_hbm.at[sl], colv[p], sem_l)
            pltpu.async_copy(w_hbm.at[sl], wv[p], sem_l)

        def wait_loads(u, p):
            sl = pl.ds(base0 + u * SUPER, SUPER)
            pltpu.make_async_copy(row_hbm.at[sl], rowv[p], sem_l).wait()
            pltpu.make_async_copy(col_hbm.at[sl], colv[p], sem_l).wait()
            pltpu.make_async_copy(w_hbm.at[sl], wv[p], sem_l).wait()

        issue_loads(0, 0)
        wait_loads(0, 0)
        issue_loads(1, 1)

        def step(u, p):
            vv = valv[p]
            for k in range(SUPER):
                for j in range(CHUNK // LANES):
                    sl = pl.ds(j * LANES, LANES)
                    wz = jnp.where(rowv[p][k, sl] == colv[p][k, sl],
                                   0.0, wv[p][k, sl])
                    for e in range(LANES):
                        vv[k * CHUNK + j * LANES + e] = (
                            jnp.where(iota == e, wz, 0.0))
                pltpu.async_copy(vv.at[pl.ds(k * CHUNK, CHUNK)],
                                 acc_sh.at[rowv[p].at[k]], sem_s, add=True)
            for k in range(SUPER):
                pltpu.make_async_copy(vv.at[pl.ds(k * CHUNK, CHUNK)],
                                      acc_sh.at[rowv[p].at[k]], sem_s).wait()
            @pl.when(u + 1 < n_steps)
            def _():
                wait_loads(u + 1, 1 - p)
            @pl.when(u + 2 < n_steps)
            def _():
                issue_loads(u + 2, p)

        def round_(r, carry):
            step(2 * r, 0)
            step(2 * r + 1, 1)
            return carry

        lax.fori_loop(0, n_steps // 2, round_, 0)
        plsc.subcore_barrier()
        pltpu.sync_copy(acc_sh.at[pl.ds(s * stripe, stripe)],
                        deg_out.at[pl.ds(c * n_pad + s * stripe, stripe)])

    return deg_kernel(row2, col2, w2, jnp.zeros((stripe, LANES), F32))


SUPER = 4
SEDGES = SUPER * CHUNK
SUPER_E = 8
SEDGES_E = SUPER_E * CHUNK


def _sc_edge(row2, col2, wz2, gp):
    t_rows = row2.shape[0]
    e_pad = t_rows * CHUNK
    n_pad, width = gp.shape
    per_tile = e_pad // NW
    n_steps = per_tile // SEDGES_E
    assert n_steps % 2 == 0 and n_steps >= 4
    stripe = n_pad // NS

    @functools.partial(
        pl.kernel,
        out_type=jax.ShapeDtypeStruct((NC * n_pad, width), F32),
        mesh=_mesh(),
        scratch_types=[
            pltpu.VMEM_SHARED((n_pad, width), F32),
            pltpu.VMEM_SHARED((n_pad, width), F32),
            pltpu.VMEM((SUPER_E, CHUNK), I32), pltpu.VMEM((SUPER_E, CHUNK), I32),
            pltpu.VMEM((SUPER_E, CHUNK), I32), pltpu.VMEM((SUPER_E, CHUNK), I32),
            pltpu.VMEM((SUPER_E, CHUNK), F32), pltpu.VMEM((SUPER_E, CHUNK), F32),
            pltpu.VMEM((SEDGES_E, width), F32), pltpu.VMEM((SEDGES_E, width), F32),
            pltpu.SemaphoreType.DMA, pltpu.SemaphoreType.DMA,
            pltpu.SemaphoreType.DMA,
        ],
        compiler_params=_SC_PARAMS,
    )
    def edge_kernel(row_hbm, col_hbm, w_hbm, gp_hbm, z_hbm, acc_out,
                    acc_sh, gp_sh, rowv0, rowv1, colv0, colv1, wv0, wv1,
                    rows0, rows1, sem_l, sem_g, sem_s):
        c = lax.axis_index("c")
        s = lax.axis_index("s")
        wid = c * NS + s
        pltpu.sync_copy(z_hbm, acc_sh.at[pl.ds(s * stripe, stripe)])
        pltpu.sync_copy(gp_hbm.at[pl.ds(s * stripe, stripe)],
                        gp_sh.at[pl.ds(s * stripe, stripe)])
        plsc.subcore_barrier()
        rowv = (rowv0, rowv1)
        colv = (colv0, colv1)
        wv = (wv0, wv1)
        rows = (rows0, rows1)
        base0 = wid * (per_tile // CHUNK)

        def issue_loads(u, p):
            sl = pl.ds(base0 + u * SUPER_E, SUPER_E)
            pltpu.async_copy(row_hbm.at[sl], rowv[p], sem_l)
            pltpu.async_copy(col_hbm.at[sl], colv[p], sem_l)
            pltpu.async_copy(w_hbm.at[sl], wv[p], sem_l)

        def wait_loads(u, p):
            sl = pl.ds(base0 + u * SUPER_E, SUPER_E)
            pltpu.make_async_copy(row_hbm.at[sl], rowv[p], sem_l).wait()
            pltpu.make_async_copy(col_hbm.at[sl], colv[p], sem_l).wait()
            pltpu.make_async_copy(w_hbm.at[sl], wv[p], sem_l).wait()

        def fire_gathers(p):
            for k in range(SUPER_E):
                pltpu.async_copy(gp_sh.at[rowv[p].at[k]],
                                 rows[p].at[pl.ds(k * CHUNK, CHUNK)], sem_g)

        def wait_gathers(p):
            for k in range(SUPER_E):
                pltpu.make_async_copy(
                    gp_sh.at[rowv[p].at[k]],
                    rows[p].at[pl.ds(k * CHUNK, CHUNK)], sem_g).wait()

        def compute_scatter(p):
            rv = rows[p]
            for k in range(SUPER_E):
                for j in range(CHUNK // LANES):
                    sl = pl.ds(j * LANES, LANES)
                    wvec = jnp.where(rowv[p][k, sl] == colv[p][k, sl],
                                     0.0, wv[p][k, sl])
                    for e in range(LANES):
                        ee = k * CHUNK + j * LANES + e
                        rv[ee] = rv[ee] * _splat(wvec, e)
                pltpu.async_copy(rv.at[pl.ds(k * CHUNK, CHUNK)],
                                 acc_sh.at[colv[p].at[k]], sem_s, add=True)
            for k in range(SUPER_E):
                pltpu.make_async_copy(rv.at[pl.ds(k * CHUNK, CHUNK)],
                                      acc_sh.at[colv[p].at[k]], sem_s).wait()

        issue_loads(0, 0)
        wait_loads(0, 0)
        fire_gathers(0)
        issue_loads(1, 1)

        def step(u, p):
            @pl.when(u + 1 < n_steps)
            def _():
                wait_loads(u + 1, 1 - p)
                fire_gathers(1 - p)
            wait_gathers(p)
            compute_scatter(p)
            @pl.when(u + 2 < n_steps)
            def _():
                issue_loads(u + 2, p)

        def round_(r, carry):
            step(2 * r, 0)
            step(2 * r + 1, 1)
            return carry

        lax.fori_loop(0, n_steps // 2, round_, 0)
        plsc.subcore_barrier()
        pltpu.sync_copy(acc_sh.at[pl.ds(s * stripe, stripe)],
                        acc_out.at[pl.ds(c * n_pad + s * stripe, stripe)])

    return edge_kernel(row2, col2, wz2, gp,
                       jnp.zeros((stripe, width), F32))


_DOT = functools.partial(
    lax.dot_general,
    precision=lax.Precision.HIGHEST,
    preferred_element_type=F32,
)
_DN = (((1,), (1,)), ((), ()))


def _tc_mm_prep(x, Wa, Wb, deg_parts, blk=1024):
    n_pad, f = x.shape
    w = Wa.shape[0]
    nc, _, lanes = deg_parts.shape

    def body(x_ref, wa_ref, wb_ref, d_ref, oa_ref, dinv_ref, gp_ref):
        xb = x_ref[...]
        deg = jnp.sum(d_ref[...], axis=(0, 2), keepdims=False)[:, None]
        pos = deg > 0.0
        dinv = jnp.where(pos, lax.rsqrt(jnp.where(pos, deg, 1.0)), 0.0)
        dinv_bc = jnp.broadcast_to(dinv, (blk, w))
        oa_ref[...] = _DOT(xb, wa_ref[...], _DN)
        dinv_ref[...] = dinv_bc
        gp_ref[...] = dinv_bc * _DOT(xb, wb_ref[...], _DN)

    return pl.pallas_call(
        body,
        grid=(n_pad // blk,),
        in_specs=[
            pl.BlockSpec((blk, f), lambda i: (i, 0)),
            pl.BlockSpec((w, f), lambda i: (0, 0)),
            pl.BlockSpec((w, f), lambda i: (0, 0)),
            pl.BlockSpec((nc, blk, lanes), lambda i: (0, i, 0)),
        ],
        out_specs=[
            pl.BlockSpec((blk, w), lambda i: (i, 0)),
            pl.BlockSpec((blk, w), lambda i: (i, 0)),
            pl.BlockSpec((blk, w), lambda i: (i, 0)),
        ],
        out_shape=[
            jax.ShapeDtypeStruct((n_pad, w), F32),
            jax.ShapeDtypeStruct((n_pad, w), F32),
            jax.ShapeDtypeStruct((n_pad, w), F32),
        ],
    )(x, Wa, Wb, deg_parts)


def _tc_fuse_mid(g1a, s1_parts, dinv_bc, b1, W2a, W2b, blk=1024):
    n_pad, hid = g1a.shape
    w2 = W2a.shape[0]

    def body(ga_ref, s_ref, dinv_ref, b_ref, wa_ref, wb_ref, oa_ref, ogp_ref):
        dinv = dinv_ref[...]
        h = ga_ref[...] - dinv * jnp.sum(s_ref[...], axis=0) + b_ref[...]
        h = jnp.maximum(h, 0.0)
        oa_ref[...] = _DOT(h, wa_ref[...], _DN)
        ogp_ref[...] = dinv * _DOT(h, wb_ref[...], _DN)

    return pl.pallas_call(
        body,
        grid=(n_pad // blk,),
        in_specs=[
            pl.BlockSpec((blk, hid), lambda i: (i, 0)),
            pl.BlockSpec((NC, blk, hid), lambda i: (0, i, 0)),
            pl.BlockSpec((blk, hid), lambda i: (i, 0)),
            pl.BlockSpec((1, hid), lambda i: (0, 0)),
            pl.BlockSpec((w2, hid), lambda i: (0, 0)),
            pl.BlockSpec((w2, hid), lambda i: (0, 0)),
        ],
        out_specs=[
            pl.BlockSpec((blk, w2), lambda i: (i, 0)),
            pl.BlockSpec((blk, w2), lambda i: (i, 0)),
        ],
        out_shape=[
            jax.ShapeDtypeStruct((n_pad, w2), F32),
            jax.ShapeDtypeStruct((n_pad, w2), F32),
        ],
    )(g1a, s1_parts, dinv_bc, b1, W2a, W2b)


def _tc_fuse_out(g2a, s2_parts, dinv_bc, b2, blk=1024):
    n_pad, ncls = g2a.shape

    def body(ga_ref, s_ref, dinv_ref, b_ref, o_ref):
        z = (ga_ref[...] - dinv_ref[...] * jnp.sum(s_ref[...], axis=0)
             + b_ref[...])
        m = jnp.max(z, axis=1, keepdims=True)
        zm = z - m
        o_ref[...] = zm - jnp.log(jnp.sum(jnp.exp(zm), axis=1, keepdims=True))

    return pl.pallas_call(
        body,
        grid=(n_pad // blk,),
        in_specs=[
            pl.BlockSpec((blk, ncls), lambda i: (i, 0)),
            pl.BlockSpec((NC, blk, ncls), lambda i: (0, i, 0)),
            pl.BlockSpec((blk, ncls), lambda i: (i, 0)),
            pl.BlockSpec((1, ncls), lambda i: (0, 0)),
        ],
        out_specs=pl.BlockSpec((blk, ncls), lambda i: (i, 0)),
        out_shape=jax.ShapeDtypeStruct((n_pad, ncls), F32),
    )(g2a, s2_parts, dinv_bc, b2)


def kernel(x, edge_index, edge_attr, W1_0, W1_1, b1, W2_0, W2_1, b2):
    n, f_in = x.shape
    e = edge_attr.shape[0]
    hid = W1_0.shape[0]
    ncls = W2_0.shape[0]

    n_pad = _round_up(n, NS * 128)
    e_pad = _round_up(e, NW * CHUNK * 16)

    row_p = jnp.pad(edge_index[0], (0, e_pad - e))
    col_p = jnp.pad(edge_index[1], (0, e_pad - e))
    w_p = jnp.pad(edge_attr, (0, e_pad - e))
    x_pad = jnp.pad(x, ((0, n_pad - n), (0, 0)))

    row2 = row_p.reshape(-1, CHUNK)
    col2 = col_p.reshape(-1, CHUNK)
    w2 = w_p.reshape(-1, CHUNK)
    deg_parts = _sc_deg(row2, col2, w2, n_pad)
    g1a, dinv_bc, gp1 = _tc_mm_prep(x_pad, W1_0, W1_1,
                                    deg_parts.reshape(NC, n_pad, LANES))

    s1_flat = _sc_edge(row2, col2, w2, gp1)
    g2a, gp2 = _tc_fuse_mid(g1a, s1_flat.reshape(NC, n_pad, hid), dinv_bc,
                            b1.reshape(1, hid), W2_0, W2_1)
    s2_flat = _sc_edge(row2, col2, w2, gp2)
    out = _tc_fuse_out(g2a, s2_flat.reshape(NC, n_pad, ncls), dinv_bc,
                       b2.reshape(1, ncls))
    return out[:n]


# speedup vs baseline: 37.8492x; 1.0115x over previous
import functools

import jax
import jax.numpy as jnp
from jax import lax
from jax.experimental import pallas as pl
from jax.experimental.pallas import tpu as pltpu
from jax.experimental.pallas import tpu_sc as plsc

NC = 2
NS = 16
NW = NC * NS
LANES = 16
CHUNK = 128

F32 = jnp.float32
I32 = jnp.int32

_SC_PARAMS = pltpu.CompilerParams(use_tc_tiling_on_sc=False)


def _round_up(v, m):
    return (v + m - 1) // m * m


def _mesh():
    return plsc.VectorSubcoreMesh(core_axis_name="c", subcore_axis_name="s",
                                  num_cores=NC, num_subcores=NS)


def _splat(vec, e):
    idx = jnp.full((LANES,), e, I32)
    return lax.gather(
        vec, idx[:, None],
        lax.GatherDimensionNumbers(offset_dims=(), collapsed_slice_dims=(0,),
                                   start_index_map=(0,)),
        (1,), mode=lax.GatherScatterMode.PROMISE_IN_BOUNDS)


def _sc_deg(row2, col2, w2, n_pad):
    t_rows = row2.shape[0]
    e_pad = t_rows * CHUNK
    per_tile = e_pad // NW
    n_steps = per_tile // SEDGES
    assert n_steps % 2 == 0 and n_steps >= 4
    stripe = n_pad // NS

    @functools.partial(
        pl.kernel,
        out_type=jax.ShapeDtypeStruct((NC * n_pad, LANES), F32),
        mesh=_mesh(),
        scratch_types=[
            pltpu.VMEM_SHARED((n_pad, LANES), F32),
            pltpu.VMEM((SUPER, CHUNK), I32), pltpu.VMEM((SUPER, CHUNK), I32),
            pltpu.VMEM((SUPER, CHUNK), I32), pltpu.VMEM((SUPER, CHUNK), I32),
            pltpu.VMEM((SUPER, CHUNK), F32), pltpu.VMEM((SUPER, CHUNK), F32),
            pltpu.VMEM((SEDGES, LANES), F32), pltpu.VMEM((SEDGES, LANES), F32),
            pltpu.SemaphoreType.DMA, pltpu.SemaphoreType.DMA,
        ],
        compiler_params=_SC_PARAMS,
    )
    def deg_kernel(row_hbm, col_hbm, w_hbm, z_hbm, deg_out,
                   acc_sh, rowv0, rowv1, colv0, colv1, wv0, wv1,
                   valv0, valv1, sem_l, sem_s):
        c = lax.axis_index("c")
        s = lax.axis_index("s")
        wid = c * NS + s
        pltpu.sync_copy(z_hbm, acc_sh.at[pl.ds(s * stripe, stripe)])
        plsc.subcore_barrier()
        rowv = (rowv0, rowv1)
        colv = (colv0, colv1)
        wv = (wv0, wv1)
        valv = (valv0, valv1)
        base0 = wid * (per_tile // CHUNK)
        iota = lax.broadcasted_iota(I32, (LANES,), 0)

        def issue_loads(u, p):
            sl = pl.ds(base0 + u * SUPER, SUPER)
            pltpu.async_copy(row_hbm.at[sl], rowv[p], sem_l)
            pltpu.async_copy(col_hbm.at[sl], colv[p], sem_l)
            pltpu.async_copy(w_hbm.at[sl], wv[p], sem_l)

        def wait_loads(u, p):
            sl = pl.ds(base0 + u * SUPER, SUPER)
            pltpu.make_async_copy(row_hbm.at[sl], rowv[p], sem_l).wait()
            pltpu.make_async_copy(col_hbm.at[sl], colv[p], sem_l).wait()
            pltpu.make_async_copy(w_hbm.at[sl], wv[p], sem_l).wait()

        issue_loads(0, 0)
        wait_loads(0, 0)
        issue_loads(1, 1)

        def step(u, p):
            vv = valv[p]
            for k in range(SUPER):
                for j in range(CHUNK // LANES):
                    sl = pl.ds(j * LANES, LANES)
                    wz = jnp.where(rowv[p][k, sl] == colv[p][k, sl],
                                   0.0, wv[p][k, sl])
                    for e in range(LANES):
                        vv[k * CHUNK + j * LANES + e] = (
                            jnp.where(iota == e, wz, 0.0))
                pltpu.async_copy(vv.at[pl.ds(k * CHUNK, CHUNK)],
                                 acc_sh.at[rowv[p].at[k]], sem_s, add=True)
            for k in range(SUPER):
                pltpu.make_async_copy(vv.at[pl.ds(k * CHUNK, CHUNK)],
                                      acc_sh.at[rowv[p].at[k]], sem_s).wait()
            @pl.when(u + 1 < n_steps)
            def _():
                wait_loads(u + 1, 1 - p)
            @pl.when(u + 2 < n_steps)
            def _():
                issue_loads(u + 2, p)

        def round_(r, carry):
            step(2 * r, 0)
            step(2 * r + 1, 1)
            return carry

        lax.fori_loop(0, n_steps // 2, round_, 0)
        plsc.subcore_barrier()
        pltpu.sync_copy(acc_sh.at[pl.ds(s * stripe, stripe)],
                        deg_out.at[pl.ds(c * n_pad + s * stripe, stripe)])

    return deg_kernel(row2, col2, w2, jnp.zeros((stripe, LANES), F32))


SUPER = 4
SEDGES = SUPER * CHUNK
SUPER_E = 4
SEDGES_E = SUPER_E * CHUNK


def _sc_edge(row2, col2, wz2, gp):
    t_rows = row2.shape[0]
    e_pad = t_rows * CHUNK
    n_pad, width = gp.shape
    per_tile = e_pad // NW
    n_steps = per_tile // SEDGES_E
    assert n_steps % 2 == 0 and n_steps >= 4
    stripe = n_pad // NS

    @functools.partial(
        pl.kernel,
        out_type=jax.ShapeDtypeStruct((NC * n_pad, width), F32),
        mesh=_mesh(),
        scratch_types=[
            pltpu.VMEM_SHARED((n_pad, width), F32),
            pltpu.VMEM_SHARED((n_pad, width), F32),
            pltpu.VMEM((SUPER_E, CHUNK), I32), pltpu.VMEM((SUPER_E, CHUNK), I32),
            pltpu.VMEM((SUPER_E, CHUNK), I32), pltpu.VMEM((SUPER_E, CHUNK), I32),
            pltpu.VMEM((SUPER_E, CHUNK), F32), pltpu.VMEM((SUPER_E, CHUNK), F32),
            pltpu.VMEM((SEDGES_E, width), F32), pltpu.VMEM((SEDGES_E, width), F32),
            pltpu.SemaphoreType.DMA, pltpu.SemaphoreType.DMA,
            pltpu.SemaphoreType.DMA,
        ],
        compiler_params=_SC_PARAMS,
    )
    def edge_kernel(row_hbm, col_hbm, w_hbm, gp_hbm, z_hbm, acc_out,
                    acc_sh, gp_sh, rowv0, rowv1, colv0, colv1, wv0, wv1,
                    rows0, rows1, sem_l, sem_g, sem_s):
        c = lax.axis_index("c")
        s = lax.axis_index("s")
        wid = c * NS + s
        pltpu.sync_copy(z_hbm, acc_sh.at[pl.ds(s * stripe, stripe)])
        pltpu.sync_copy(gp_hbm.at[pl.ds(s * stripe, stripe)],
                        gp_sh.at[pl.ds(s * stripe, stripe)])
        plsc.subcore_barrier()
        rowv = (rowv0, rowv1)
        colv = (colv0, colv1)
        wv = (wv0, wv1)
        rows = (rows0, rows1)
        base0 = wid * (per_tile // CHUNK)

        def issue_loads(u, p):
            sl = pl.ds(base0 + u * SUPER_E, SUPER_E)
            pltpu.async_copy(row_hbm.at[sl], rowv[p], sem_l)
            pltpu.async_copy(col_hbm.at[sl], colv[p], sem_l)
            pltpu.async_copy(w_hbm.at[sl], wv[p], sem_l)

        def wait_loads(u, p):
            sl = pl.ds(base0 + u * SUPER_E, SUPER_E)
            pltpu.make_async_copy(row_hbm.at[sl], rowv[p], sem_l).wait()
            pltpu.make_async_copy(col_hbm.at[sl], colv[p], sem_l).wait()
            pltpu.make_async_copy(w_hbm.at[sl], wv[p], sem_l).wait()

        def fire_gathers(p):
            for k in range(SUPER_E):
                pltpu.async_copy(gp_sh.at[rowv[p].at[k]],
                                 rows[p].at[pl.ds(k * CHUNK, CHUNK)], sem_g)

        def wait_gathers(p):
            for k in range(SUPER_E):
                pltpu.make_async_copy(
                    gp_sh.at[rowv[p].at[k]],
                    rows[p].at[pl.ds(k * CHUNK, CHUNK)], sem_g).wait()

        def compute_scatter(p):
            rv = rows[p]
            for k in range(SUPER_E):
                for j in range(CHUNK // LANES):
                    sl = pl.ds(j * LANES, LANES)
                    wvec = jnp.where(rowv[p][k, sl] == colv[p][k, sl],
                                     0.0, wv[p][k, sl])
                    for e in range(LANES):
                        ee = k * CHUNK + j * LANES + e
                        rv[ee] = rv[ee] * _splat(wvec, e)
                pltpu.async_copy(rv.at[pl.ds(k * CHUNK, CHUNK)],
                                 acc_sh.at[colv[p].at[k]], sem_s, add=True)
            for k in range(SUPER_E):
                pltpu.make_async_copy(rv.at[pl.ds(k * CHUNK, CHUNK)],
                                      acc_sh.at[colv[p].at[k]], sem_s).wait()

        issue_loads(0, 0)
        wait_loads(0, 0)
        fire_gathers(0)
        issue_loads(1, 1)

        def step(u, p):
            @pl.when(u + 1 < n_steps)
            def _():
                wait_loads(u + 1, 1 - p)
                fire_gathers(1 - p)
            wait_gathers(p)
            compute_scatter(p)
            @pl.when(u + 2 < n_steps)
            def _():
                issue_loads(u + 2, p)

        def round_(r, carry):
            step(2 * r, 0)
            step(2 * r + 1, 1)
            return carry

        lax.fori_loop(0, n_steps // 2, round_, 0)
        plsc.subcore_barrier()
        pltpu.sync_copy(acc_sh.at[pl.ds(s * stripe, stripe)],
                        acc_out.at[pl.ds(c * n_pad + s * stripe, stripe)])

    return edge_kernel(row2, col2, wz2, gp,
                       jnp.zeros((stripe, width), F32))


_DOT = functools.partial(
    lax.dot_general,
    precision=lax.Precision.HIGHEST,
    preferred_element_type=F32,
)
_DN = (((1,), (1,)), ((), ()))


def _tc_mm_prep(x, Wa, Wb, deg_parts, blk=1024):
    n_pad, f = x.shape
    w = Wa.shape[0]
    nc, _, lanes = deg_parts.shape

    def body(x_ref, wa_ref, wb_ref, d_ref, oa_ref, dinv_ref, gp_ref):
        xb = x_ref[...]
        deg = jnp.sum(d_ref[...], axis=(0, 2), keepdims=False)[:, None]
        pos = deg > 0.0
        dinv = jnp.where(pos, lax.rsqrt(jnp.where(pos, deg, 1.0)), 0.0)
        dinv_bc = jnp.broadcast_to(dinv, (blk, w))
        oa_ref[...] = _DOT(xb, wa_ref[...], _DN)
        dinv_ref[...] = dinv_bc
        gp_ref[...] = dinv_bc * _DOT(xb, wb_ref[...], _DN)

    return pl.pallas_call(
        body,
        grid=(n_pad // blk,),
        in_specs=[
            pl.BlockSpec((blk, f), lambda i: (i, 0)),
            pl.BlockSpec((w, f), lambda i: (0, 0)),
            pl.BlockSpec((w, f), lambda i: (0, 0)),
            pl.BlockSpec((nc, blk, lanes), lambda i: (0, i, 0)),
        ],
        out_specs=[
            pl.BlockSpec((blk, w), lambda i: (i, 0)),
            pl.BlockSpec((blk, w), lambda i: (i, 0)),
            pl.BlockSpec((blk, w), lambda i: (i, 0)),
        ],
        out_shape=[
            jax.ShapeDtypeStruct((n_pad, w), F32),
            jax.ShapeDtypeStruct((n_pad, w), F32),
            jax.ShapeDtypeStruct((n_pad, w), F32),
        ],
    )(x, Wa, Wb, deg_parts)


def _tc_fuse_mid(g1a, s1_parts, dinv_bc, b1, W2a, W2b, blk=1024):
    n_pad, hid = g1a.shape
    w2 = W2a.shape[0]

    def body(ga_ref, s_ref, dinv_ref, b_ref, wa_ref, wb_ref, oa_ref, ogp_ref):
        dinv = dinv_ref[...]
        h = ga_ref[...] - dinv * jnp.sum(s_ref[...], axis=0) + b_ref[...]
        h = jnp.maximum(h, 0.0)
        oa_ref[...] = _DOT(h, wa_ref[...], _DN)
        ogp_ref[...] = dinv * _DOT(h, wb_ref[...], _DN)

    return pl.pallas_call(
        body,
        grid=(n_pad // blk,),
        in_specs=[
            pl.BlockSpec((blk, hid), lambda i: (i, 0)),
            pl.BlockSpec((NC, blk, hid), lambda i: (0, i, 0)),
            pl.BlockSpec((blk, hid), lambda i: (i, 0)),
            pl.BlockSpec((1, hid), lambda i: (0, 0)),
            pl.BlockSpec((w2, hid), lambda i: (0, 0)),
            pl.BlockSpec((w2, hid), lambda i: (0, 0)),
        ],
        out_specs=[
            pl.BlockSpec((blk, w2), lambda i: (i, 0)),
            pl.BlockSpec((blk, w2), lambda i: (i, 0)),
        ],
        out_shape=[
            jax.ShapeDtypeStruct((n_pad, w2), F32),
            jax.ShapeDtypeStruct((n_pad, w2), F32),
        ],
    )(g1a, s1_parts, dinv_bc, b1, W2a, W2b)


def _tc_fuse_out(g2a, s2_parts, dinv_bc, b2, blk=1024):
    n_pad, ncls = g2a.shape

    def body(ga_ref, s_ref, dinv_ref, b_ref, o_ref):
        z = (ga_ref[...] - dinv_ref[...] * jnp.sum(s_ref[...], axis=0)
             + b_ref[...])
        m = jnp.max(z, axis=1, keepdims=True)
        zm = z - m
        o_ref[...] = zm - jnp.log(jnp.sum(jnp.exp(zm), axis=1, keepdims=True))

    return pl.pallas_call(
        body,
        grid=(n_pad // blk,),
        in_specs=[
            pl.BlockSpec((blk, ncls), lambda i: (i, 0)),
            pl.BlockSpec((NC, blk, ncls), lambda i: (0, i, 0)),
            pl.BlockSpec((blk, ncls), lambda i: (i, 0)),
            pl.BlockSpec((1, ncls), lambda i: (0, 0)),
        ],
        out_specs=pl.BlockSpec((blk, ncls), lambda i: (i, 0)),
        out_shape=jax.ShapeDtypeStruct((n_pad, ncls), F32),
    )(g2a, s2_parts, dinv_bc, b2)


def kernel(x, edge_index, edge_attr, W1_0, W1_1, b1, W2_0, W2_1, b2):
    n, f_in = x.shape
    e = edge_attr.shape[0]
    hid = W1_0.shape[0]
    ncls = W2_0.shape[0]

    n_pad = _round_up(n, NS * 128)
    e_pad = _round_up(e, NW * CHUNK * 16)

    row_p = jnp.pad(edge_index[0], (0, e_pad - e))
    col_p = jnp.pad(edge_index[1], (0, e_pad - e))
    w_p = jnp.pad(edge_attr, (0, e_pad - e))
    x_pad = jnp.pad(x, ((0, n_pad - n), (0, 0)))

    row2 = row_p.reshape(-1, CHUNK)
    col2 = col_p.reshape(-1, CHUNK)
    w2 = w_p.reshape(-1, CHUNK)
    deg_parts = _sc_deg(row2, col2, w2, n_pad)
    g1a, dinv_bc, gp1 = _tc_mm_prep(x_pad, W1_0, W1_1,
                                    deg_parts.reshape(NC, n_pad, LANES))

    s1_flat = _sc_edge(row2, col2, w2, gp1)
    g2a, gp2 = _tc_fuse_mid(g1a, s1_flat.reshape(NC, n_pad, hid), dinv_bc,
                            b1.reshape(1, hid), W2_0, W2_1)
    s2_flat = _sc_edge(row2, col2, w2, gp2)
    out = _tc_fuse_out(g2a, s2_flat.reshape(NC, n_pad, ncls), dinv_bc,
                       b2.reshape(1, ncls))
    return out[:n]


# speedup vs baseline: 37.9621x; 1.0030x over previous
import functools

import jax
import jax.numpy as jnp
from jax import lax
from jax.experimental import pallas as pl
from jax.experimental.pallas import tpu as pltpu
from jax.experimental.pallas import tpu_sc as plsc

NC = 2
NS = 16
NW = NC * NS
LANES = 16
CHUNK = 128

F32 = jnp.float32
I32 = jnp.int32

_SC_PARAMS = pltpu.CompilerParams(use_tc_tiling_on_sc=False)


def _round_up(v, m):
    return (v + m - 1) // m * m


def _mesh():
    return plsc.VectorSubcoreMesh(core_axis_name="c", subcore_axis_name="s",
                                  num_cores=NC, num_subcores=NS)


def _splat(vec, e):
    idx = jnp.full((LANES,), e, I32)
    return lax.gather(
        vec, idx[:, None],
        lax.GatherDimensionNumbers(offset_dims=(), collapsed_slice_dims=(0,),
                                   start_index_map=(0,)),
        (1,), mode=lax.GatherScatterMode.PROMISE_IN_BOUNDS)


def _sc_deg(row2, col2, w2, n_pad):
    t_rows = row2.shape[0]
    e_pad = t_rows * CHUNK
    per_tile = e_pad // NW
    n_steps = per_tile // SEDGES
    assert n_steps % 2 == 0 and n_steps >= 4
    stripe = n_pad // NS

    @functools.partial(
        pl.kernel,
        out_type=jax.ShapeDtypeStruct((NC * n_pad, LANES), F32),
        mesh=_mesh(),
        scratch_types=[
            pltpu.VMEM_SHARED((n_pad, LANES), F32),
            pltpu.VMEM((SUPER, CHUNK), I32), pltpu.VMEM((SUPER, CHUNK), I32),
            pltpu.VMEM((SUPER, CHUNK), I32), pltpu.VMEM((SUPER, CHUNK), I32),
            pltpu.VMEM((SUPER, CHUNK), F32), pltpu.VMEM((SUPER, CHUNK), F32),
            pltpu.VMEM((SEDGES, LANES), F32), pltpu.VMEM((SEDGES, LANES), F32),
            pltpu.SemaphoreType.DMA, pltpu.SemaphoreType.DMA,
        ],
        compiler_params=_SC_PARAMS,
    )
    def deg_kernel(row_hbm, col_hbm, w_hbm, z_hbm, deg_out,
                   acc_sh, rowv0, rowv1, colv0, colv1, wv0, wv1,
                   valv0, valv1, sem_l, sem_s):
        c = lax.axis_index("c")
        s = lax.axis_index("s")
        wid = c * NS + s
        pltpu.sync_copy(z_hbm, acc_sh.at[pl.ds(s * stripe, stripe)])
        plsc.subcore_barrier()
        rowv = (rowv0, rowv1)
        colv = (colv0, colv1)
        wv = (wv0, wv1)
        valv = (valv0, valv1)
        base0 = wid * (per_tile // CHUNK)
        iota = lax.broadcasted_iota(I32, (LANES,), 0)

        def issue_loads(u, p):
            sl = pl.ds(base0 + u * SUPER, SUPER)
            pltpu.async_copy(row_hbm.at[sl], rowv[p], sem_l)
            pltpu.async_copy(col_hbm.at[sl], colv[p], sem_l)
            pltpu.async_copy(w_hbm.at[sl], wv[p], sem_l)

        def wait_loads(u, p):
            sl = pl.ds(base0 + u * SUPER, SUPER)
            pltpu.make_async_copy(row_hbm.at[sl], rowv[p], sem_l).wait()
            pltpu.make_async_copy(col_hbm.at[sl], colv[p], sem_l).wait()
            pltpu.make_async_copy(w_hbm.at[sl], wv[p], sem_l).wait()

        issue_loads(0, 0)
        wait_loads(0, 0)
        issue_loads(1, 1)

        def step(u, p):
            vv = valv[p]
            for k in range(SUPER):
                for j in range(CHUNK // LANES):
                    sl = pl.ds(j * LANES, LANES)
                    wz = jnp.where(rowv[p][k, sl] == colv[p][k, sl],
                                   0.0, wv[p][k, sl])
                    for e in range(LANES):
                        vv[k * CHUNK + j * LANES + e] = (
                            jnp.where(iota == e, wz, 0.0))
                pltpu.async_copy(vv.at[pl.ds(k * CHUNK, CHUNK)],
                                 acc_sh.at[rowv[p].at[k]], sem_s, add=True)
            for k in range(SUPER):
                pltpu.make_async_copy(vv.at[pl.ds(k * CHUNK, CHUNK)],
                                      acc_sh.at[rowv[p].at[k]], sem_s).wait()
            @pl.when(u + 1 < n_steps)
            def _():
                wait_loads(u + 1, 1 - p)
            @pl.when(u + 2 < n_steps)
            def _():
                issue_loads(u + 2, p)

        def round_(r, carry):
            step(2 * r, 0)
            step(2 * r + 1, 1)
            return carry

        lax.fori_loop(0, n_steps // 2, round_, 0)
        plsc.subcore_barrier()
        pltpu.sync_copy(acc_sh.at[pl.ds(s * stripe, stripe)],
                        deg_out.at[pl.ds(c * n_pad + s * stripe, stripe)])

    return deg_kernel(row2, col2, w2, jnp.zeros((stripe, LANES), F32))


SUPER = 4
SEDGES = SUPER * CHUNK
SUPER_E = 4
SEDGES_E = SUPER_E * CHUNK


def _sc_edge(row2, col2, w2, gp):
    t_rows = row2.shape[0]
    e_pad = t_rows * CHUNK
    n_pad, width = gp.shape
    per_tile = e_pad // NW
    n_steps = per_tile // SEDGES_E
    assert n_steps % 2 == 0 and n_steps >= 4
    stripe = n_pad // NS

    @functools.partial(
        pl.kernel,
        out_type=jax.ShapeDtypeStruct((NC * n_pad, width), F32),
        mesh=_mesh(),
        scratch_types=[
            pltpu.VMEM_SHARED((n_pad, width), F32),
            pltpu.VMEM_SHARED((n_pad, width), F32),
            pltpu.VMEM((SUPER_E, CHUNK), I32), pltpu.VMEM((SUPER_E, CHUNK), I32),
            pltpu.VMEM((SUPER_E, CHUNK), I32), pltpu.VMEM((SUPER_E, CHUNK), I32),
            pltpu.VMEM((SUPER_E, CHUNK), F32), pltpu.VMEM((SUPER_E, CHUNK), F32),
            pltpu.VMEM((SEDGES_E, width), F32), pltpu.VMEM((SEDGES_E, width), F32),
            pltpu.SemaphoreType.DMA, pltpu.SemaphoreType.DMA,
            pltpu.SemaphoreType.DMA,
        ],
        compiler_params=_SC_PARAMS,
    )
    def edge_kernel(row_hbm, col_hbm, w_hbm, gp_hbm, z_hbm, acc_out,
                    acc_sh, gp_sh, rowv0, rowv1, colv0, colv1, wv0, wv1,
                    rows0, rows1, sem_l, sem_g, sem_s):
        c = lax.axis_index("c")
        s = lax.axis_index("s")
        wid = c * NS + s
        pltpu.sync_copy(z_hbm, acc_sh.at[pl.ds(s * stripe, stripe)])
        pltpu.sync_copy(gp_hbm.at[pl.ds(s * stripe, stripe)],
                        gp_sh.at[pl.ds(s * stripe, stripe)])
        plsc.subcore_barrier()
        rowv = (rowv0, rowv1)
        colv = (colv0, colv1)
        wv = (wv0, wv1)
        rows = (rows0, rows1)
        base0 = wid * (per_tile // CHUNK)

        def issue_loads(u, p):
            sl = pl.ds(base0 + u * SUPER_E, SUPER_E)
            pltpu.async_copy(row_hbm.at[sl], rowv[p], sem_l)
            pltpu.async_copy(col_hbm.at[sl], colv[p], sem_l)
            pltpu.async_copy(w_hbm.at[sl], wv[p], sem_l)

        def wait_loads(u, p):
            sl = pl.ds(base0 + u * SUPER_E, SUPER_E)
            pltpu.make_async_copy(row_hbm.at[sl], rowv[p], sem_l).wait()
            pltpu.make_async_copy(col_hbm.at[sl], colv[p], sem_l).wait()
            pltpu.make_async_copy(w_hbm.at[sl], wv[p], sem_l).wait()

        def fire_gathers(p):
            for k in range(SUPER_E):
                pltpu.async_copy(gp_sh.at[rowv[p].at[k]],
                                 rows[p].at[pl.ds(k * CHUNK, CHUNK)], sem_g)

        def wait_gathers(p):
            for k in range(SUPER_E):
                pltpu.make_async_copy(
                    gp_sh.at[rowv[p].at[k]],
                    rows[p].at[pl.ds(k * CHUNK, CHUNK)], sem_g).wait()

        def compute_scatter(p):
            rv = rows[p]
            for k in range(SUPER_E):
                for j in range(CHUNK // LANES):
                    sl = pl.ds(j * LANES, LANES)
                    wvec = jnp.where(rowv[p][k, sl] == colv[p][k, sl],
                                     0.0, wv[p][k, sl])
                    for e in range(LANES):
                        ee = k * CHUNK + j * LANES + e
                        rv[ee] = rv[ee] * _splat(wvec, e)
                pltpu.async_copy(rv.at[pl.ds(k * CHUNK, CHUNK)],
                                 acc_sh.at[colv[p].at[k]], sem_s, add=True)
            for k in range(SUPER_E):
                pltpu.make_async_copy(rv.at[pl.ds(k * CHUNK, CHUNK)],
                                      acc_sh.at[colv[p].at[k]], sem_s).wait()

        issue_loads(0, 0)
        wait_loads(0, 0)
        fire_gathers(0)
        issue_loads(1, 1)

        def step(u, p):
            @pl.when(u + 1 < n_steps)
            def _():
                wait_loads(u + 1, 1 - p)
                fire_gathers(1 - p)
            wait_gathers(p)
            compute_scatter(p)
            @pl.when(u + 2 < n_steps)
            def _():
                issue_loads(u + 2, p)

        def round_(r, carry):
            step(2 * r, 0)
            step(2 * r + 1, 1)
            return carry

        lax.fori_loop(0, n_steps // 2, round_, 0)
        plsc.subcore_barrier()
        pltpu.sync_copy(acc_sh.at[pl.ds(s * stripe, stripe)],
                        acc_out.at[pl.ds(c * n_pad + s * stripe, stripe)])

    return edge_kernel(row2, col2, w2, gp,
                       jnp.zeros((stripe, width), F32))


_DOT = functools.partial(
    lax.dot_general,
    precision=lax.Precision.HIGHEST,
    preferred_element_type=F32,
)
_DN = (((1,), (1,)), ((), ()))


def _tc_mm_prep(x, Wa, Wb, deg_parts, blk=1024):
    n_pad, f = x.shape
    w = Wa.shape[0]
    nc, _, lanes = deg_parts.shape

    def body(x_ref, wa_ref, wb_ref, d_ref, oa_ref, dinv_ref, gp_ref):
        xb = x_ref[...]
        deg = jnp.sum(d_ref[...], axis=(0, 2), keepdims=False)[:, None]
        pos = deg > 0.0
        dinv = jnp.where(pos, lax.rsqrt(jnp.where(pos, deg, 1.0)), 0.0)
        dinv_bc = jnp.broadcast_to(dinv, (blk, w))
        oa_ref[...] = _DOT(xb, wa_ref[...], _DN)
        dinv_ref[...] = dinv_bc
        gp_ref[...] = dinv_bc * _DOT(xb, wb_ref[...], _DN)

    return pl.pallas_call(
        body,
        grid=(n_pad // blk,),
        in_specs=[
            pl.BlockSpec((blk, f), lambda i: (i, 0)),
            pl.BlockSpec((w, f), lambda i: (0, 0)),
            pl.BlockSpec((w, f), lambda i: (0, 0)),
            pl.BlockSpec((nc, blk, lanes), lambda i: (0, i, 0)),
        ],
        out_specs=[
            pl.BlockSpec((blk, w), lambda i: (i, 0)),
            pl.BlockSpec((blk, w), lambda i: (i, 0)),
            pl.BlockSpec((blk, w), lambda i: (i, 0)),
        ],
        out_shape=[
            jax.ShapeDtypeStruct((n_pad, w), F32),
            jax.ShapeDtypeStruct((n_pad, w), F32),
            jax.ShapeDtypeStruct((n_pad, w), F32),
        ],
    )(x, Wa, Wb, deg_parts)


def _tc_fuse_mid(g1a, s1_parts, dinv_bc, b1, W2a, W2b, blk=1024):
    n_pad, hid = g1a.shape
    w2 = W2a.shape[0]

    def body(ga_ref, s_ref, dinv_ref, b_ref, wa_ref, wb_ref, oa_ref, ogp_ref):
        dinv = dinv_ref[...]
        h = ga_ref[...] - dinv * jnp.sum(s_ref[...], axis=0) + b_ref[...]
        h = jnp.maximum(h, 0.0)
        oa_ref[...] = _DOT(h, wa_ref[...], _DN)
        ogp_ref[...] = dinv * _DOT(h, wb_ref[...], _DN)

    return pl.pallas_call(
        body,
        grid=(n_pad // blk,),
        in_specs=[
            pl.BlockSpec((blk, hid), lambda i: (i, 0)),
            pl.BlockSpec((NC, blk, hid), lambda i: (0, i, 0)),
            pl.BlockSpec((blk, hid), lambda i: (i, 0)),
            pl.BlockSpec((1, hid), lambda i: (0, 0)),
            pl.BlockSpec((w2, hid), lambda i: (0, 0)),
            pl.BlockSpec((w2, hid), lambda i: (0, 0)),
        ],
        out_specs=[
            pl.BlockSpec((blk, w2), lambda i: (i, 0)),
            pl.BlockSpec((blk, w2), lambda i: (i, 0)),
        ],
        out_shape=[
            jax.ShapeDtypeStruct((n_pad, w2), F32),
            jax.ShapeDtypeStruct((n_pad, w2), F32),
        ],
    )(g1a, s1_parts, dinv_bc, b1, W2a, W2b)


def _tc_fuse_out(g2a, s2_parts, dinv_bc, b2, blk=1024):
    n_pad, ncls = g2a.shape

    def body(ga_ref, s_ref, dinv_ref, b_ref, o_ref):
        z = (ga_ref[...] - dinv_ref[...] * jnp.sum(s_ref[...], axis=0)
             + b_ref[...])
        m = jnp.max(z, axis=1, keepdims=True)
        zm = z - m
        o_ref[...] = zm - jnp.log(jnp.sum(jnp.exp(zm), axis=1, keepdims=True))

    return pl.pallas_call(
        body,
        grid=(n_pad // blk,),
        in_specs=[
            pl.BlockSpec((blk, ncls), lambda i: (i, 0)),
            pl.BlockSpec((NC, blk, ncls), lambda i: (0, i, 0)),
            pl.BlockSpec((blk, ncls), lambda i: (i, 0)),
            pl.BlockSpec((1, ncls), lambda i: (0, 0)),
        ],
        out_specs=pl.BlockSpec((blk, ncls), lambda i: (i, 0)),
        out_shape=jax.ShapeDtypeStruct((n_pad, ncls), F32),
    )(g2a, s2_parts, dinv_bc, b2)


def kernel(x, edge_index, edge_attr, W1_0, W1_1, b1, W2_0, W2_1, b2):
    n, f_in = x.shape
    e = edge_attr.shape[0]
    hid = W1_0.shape[0]
    ncls = W2_0.shape[0]

    n_pad = _round_up(n, NS * 128)
    e_pad = _round_up(e, NW * CHUNK * 16)

    row_p = jnp.pad(edge_index[0], (0, e_pad - e))
    col_p = jnp.pad(edge_index[1], (0, e_pad - e))
    w_p = jnp.pad(edge_attr, (0, e_pad - e))
    x_pad = jnp.pad(x, ((0, n_pad - n), (0, 0)))

    row2 = row_p.reshape(-1, CHUNK)
    col2 = col_p.reshape(-1, CHUNK)
    w2 = w_p.reshape(-1, CHUNK)
    deg_parts = _sc_deg(row2, col2, w2, n_pad)
    g1a, dinv_bc, gp1 = _tc_mm_prep(x_pad, W1_0, W1_1,
                                    deg_parts.reshape(NC, n_pad, LANES))

    s1_flat = _sc_edge(row2, col2, w2, gp1)
    g2a, gp2 = _tc_fuse_mid(g1a, s1_flat.reshape(NC, n_pad, hid), dinv_bc,
                            b1.reshape(1, hid), W2_0, W2_1)
    s2_flat = _sc_edge(row2, col2, w2, gp2)
    out = _tc_fuse_out(g2a, s2_flat.reshape(NC, n_pad, ncls), dinv_bc,
                       b2.reshape(1, ncls))
    return out[:n]
